# Initial kernel scaffold; baseline (speedup 1.0000x reference)
import jax
import jax.numpy as jnp
from jax.experimental import pallas as pl


def kernel(x, edge_index, W_in, b_in, W_gat, a_gat, ln1_s, ln1_b, Wqkv, bqkv, Wout, bout, ln2_s, ln2_b, Wc, bc):
    raise NotImplementedError("write your pallas kernel here")


# speedup vs baseline: 36.7039x; 36.7039x over previous
import functools
import math

import jax
import jax.numpy as jnp
from jax import lax
from jax.experimental import pallas as pl
from jax.experimental.pallas import tpu as pltpu
from jax.experimental.pallas import tpu_sc as plsc

N = 10000
E = 160000
T = 4
IN_DIM = 128
HID = 128
H = 8
DH = 16
NCLS = 40

NWORK = 32
EPAD = 163840
EW = EPAD // NWORK
CH1 = 1280
CH2 = 128
BLK = 1000
NB = N // BLK
RPT1 = 2504
TNPAD = 16 * RPT1
RPT2 = 632
NPAD = 16 * RPT2

_f32 = jnp.float32
_i32 = jnp.int32
_GATHER_DNUMS = lax.GatherDimensionNumbers(
    offset_dims=(), collapsed_slice_dims=(0,), start_index_map=(0,))


def _tca_body(x_ref, winT_ref, b_in_ref, wgatT_ref, wsrc_ref, wdst_ref,
              h0_ref, hh_ref, ss_ref, sd_ref):
    xb = x_ref[0]
    h0 = jnp.dot(xb, winT_ref[...], preferred_element_type=_f32) + b_in_ref[0]
    hh = jnp.dot(h0, wgatT_ref[...], preferred_element_type=_f32)
    ss = jnp.dot(h0, wsrc_ref[...], preferred_element_type=_f32)
    sd = jnp.dot(h0, wdst_ref[...], preferred_element_type=_f32)
    h0_ref[0] = h0
    hh_ref[0] = hh
    ss_ref[0] = ss
    sd_ref[0] = sd


def _run_tca(x, winT, b_in2, wgatT, wsrc, wdst):
    full = lambda s: pl.BlockSpec(s, lambda t, nb: tuple(0 for _ in s))
    return pl.pallas_call(
        _tca_body,
        grid=(T, NB),
        in_specs=[
            pl.BlockSpec((1, BLK, IN_DIM), lambda t, nb: (t, nb, 0)),
            full((IN_DIM, HID)),
            full((1, HID)),
            full((HID, HID)),
            full((HID, 16)),
            full((HID, 16)),
        ],
        out_specs=[
            pl.BlockSpec((1, BLK, HID), lambda t, nb: (t, nb, 0)),
            pl.BlockSpec((1, BLK, HID), lambda t, nb: (t, nb, 0)),
            pl.BlockSpec((1, BLK, 16), lambda t, nb: (t, nb, 0)),
            pl.BlockSpec((1, BLK, 16), lambda t, nb: (t, nb, 0)),
        ],
        out_shape=[
            jax.ShapeDtypeStruct((T, N, HID), _f32),
            jax.ShapeDtypeStruct((T, N, HID), _f32),
            jax.ShapeDtypeStruct((T, N, 16), _f32),
            jax.ShapeDtypeStruct((T, N, 16), _f32),
        ],
    )(x, winT, b_in2, wgatT, wsrc, wdst)


def _sc1_body(ei_ref, ssrc_ref, sdst_ref, c_ref, den_ref,
              src_off, dst_off, srow, drow, exb, cbuf, denom_sp, sem):
    cid = lax.axis_index("c")
    sid = lax.axis_index("s")
    wid = sid * 2 + cid
    pltpu.sync_copy(c_ref, cbuf)

    zv = jnp.zeros((16,), _f32)

    def zbuf(g, _):
        exb[g] = zv
        return 0

    lax.fori_loop(0, CH1, zbuf, 0)
    pltpu.sync_copy(exb, denom_sp.at[pl.ds(sid * RPT1, CH1)])
    pltpu.sync_copy(exb.at[pl.ds(0, RPT1 - CH1)],
                    denom_sp.at[pl.ds(sid * RPT1 + CH1, RPT1 - CH1)])
    plsc.subcore_barrier()

    for t in range(T):
        cvec = cbuf[t]

        def chunk(j, _, t=t, cvec=cvec):
            base = wid * EW + j * CH1
            pltpu.sync_copy(ei_ref.at[pl.ds((t * 2) * EPAD + base, CH1)],
                            src_off)
            pltpu.sync_copy(ei_ref.at[pl.ds((t * 2 + 1) * EPAD + base, CH1)],
                            dst_off)

            def addoff(g, _):
                src_off[pl.ds(g * 16, 16)] = src_off[pl.ds(g * 16, 16)] + t * N
                dst_off[pl.ds(g * 16, 16)] = dst_off[pl.ds(g * 16, 16)] + t * N
                return 0

            lax.fori_loop(0, CH1 // 16, addoff, 0)
            cp1 = pltpu.async_copy(ssrc_ref.at[src_off], srow, sem)
            cp2 = pltpu.async_copy(sdst_ref.at[dst_off], drow, sem)
            cp1.wait()
            cp2.wait()

            def body(e, _):
                s = srow[e] + drow[e]
                s = jnp.where(s > 0, s, 0.2 * s) - cvec
                ex = jnp.exp(s)
                valid = jnp.where(base + e < E, 1.0, 0.0).astype(_f32)
                exb[e] = ex * valid
                return 0

            lax.fori_loop(0, CH1, body, 0)
            pltpu.sync_copy(exb, denom_sp.at[dst_off], add=True)
            return 0

        lax.fori_loop(0, EW // CH1, chunk, 0)

    plsc.subcore_barrier()
    pltpu.sync_copy(
        denom_sp.at[pl.ds(sid * RPT1, RPT1)],
        den_ref.at[pl.ds(cid * TNPAD + sid * RPT1, RPT1)])


def _run_sc1(ei_flat, ssrc16, sdst16, cmax):
    mesh = plsc.VectorSubcoreMesh(core_axis_name="c", subcore_axis_name="s")
    k = pl.kernel(
        _sc1_body,
        out_type=jax.ShapeDtypeStruct((2 * TNPAD, 16), _f32),
        mesh=mesh,
        scratch_types=[
            pltpu.VMEM((CH1,), _i32),
            pltpu.VMEM((CH1,), _i32),
            pltpu.VMEM((CH1, 16), _f32),
            pltpu.VMEM((CH1, 16), _f32),
            pltpu.VMEM((CH1, 16), _f32),
            pltpu.VMEM((T, 16), _f32),
            pltpu.VMEM_SHARED((TNPAD, 16), _f32),
            pltpu.SemaphoreType.DMA,
        ],
        compiler_params=pltpu.CompilerParams(use_tc_tiling_on_sc=False),
    )
    return k(ei_flat, ssrc16, sdst16, cmax)


def _sc2_body(ei_ref, ssrc_ref, sdst_ref, c_ref, hh_ref, dA_ref, dB_ref,
              msg_ref, src_off, dst_loc, dst_off, hhb, srow, drow, dab, dbb,
              cbuf, out_sp, sem):
    cid = lax.axis_index("c")
    sid = lax.axis_index("s")
    wid = sid * 2 + cid
    pltpu.sync_copy(c_ref, cbuf)
    zv = jnp.zeros((16,), _f32)
    splats = [jnp.full((16,), h, _i32) for h in range(H)]

    for t in range(T):
        cvec = cbuf[t]

        def zbuf(g, _):
            for kk in range(H):
                hhb[g, pl.ds(kk * 16, 16)] = zv
            return 0

        lax.fori_loop(0, CH2, zbuf, 0)
        for z in range(RPT2 // CH2):
            pltpu.sync_copy(hhb, out_sp.at[pl.ds(sid * RPT2 + z * CH2, CH2)])
        zrem = RPT2 - (RPT2 // CH2) * CH2
        if zrem:
            pltpu.sync_copy(
                hhb.at[pl.ds(0, zrem)],
                out_sp.at[pl.ds(sid * RPT2 + (RPT2 // CH2) * CH2, zrem)])
        plsc.subcore_barrier()

        def chunk(j, _, t=t, cvec=cvec):
            base = wid * EW + j * CH2
            pltpu.sync_copy(ei_ref.at[pl.ds((t * 2) * EPAD + base, CH2)],
                            src_off)
            pltpu.sync_copy(ei_ref.at[pl.ds((t * 2 + 1) * EPAD + base, CH2)],
                            dst_loc)

            def addoff(g, _):
                src_off[pl.ds(g * 16, 16)] = src_off[pl.ds(g * 16, 16)] + t * N
                dst_off[pl.ds(g * 16, 16)] = dst_loc[pl.ds(g * 16, 16)] + t * N
                return 0

            lax.fori_loop(0, CH2 // 16, addoff, 0)
            cps = [
                pltpu.async_copy(hh_ref.at[src_off], hhb, sem),
                pltpu.async_copy(ssrc_ref.at[src_off], srow, sem),
                pltpu.async_copy(sdst_ref.at[dst_off], drow, sem),
                pltpu.async_copy(dA_ref.at[dst_off], dab, sem),
                pltpu.async_copy(dB_ref.at[dst_off], dbb, sem),
            ]
            for cp in cps:
                cp.wait()

            def body(e, _):
                s = srow[e] + drow[e]
                s = jnp.where(s > 0, s, 0.2 * s) - cvec
                ex = jnp.exp(s)
                den = dab[e] + dbb[e] + 1e-16
                valid = jnp.where(base + e < E, 1.0, 0.0).astype(_f32)
                al = ex / den * valid
                for h in range(H):
                    av = lax.gather(
                        al, splats[h][:, None], _GATHER_DNUMS, (1,),
                        mode=lax.GatherScatterMode.PROMISE_IN_BOUNDS)
                    hv = hhb[e, pl.ds(h * 16, 16)]
                    hhb[e, pl.ds(h * 16, 16)] = hv * av
                return 0

            lax.fori_loop(0, CH2, body, 0)
            pltpu.sync_copy(hhb, out_sp.at[dst_loc], add=True)
            return 0

        lax.fori_loop(0, EW // CH2, chunk, 0)
        plsc.subcore_barrier()
        pltpu.sync_copy(
            out_sp.at[pl.ds(sid * RPT2, RPT2)],
            msg_ref.at[pl.ds(cid * (T * NPAD) + t * NPAD + sid * RPT2,
                             RPT2)])
        plsc.subcore_barrier()


def _run_sc2(ei_flat, ssrc16, sdst16, cmax, hh_flat, dA, dB):
    mesh = plsc.VectorSubcoreMesh(core_axis_name="c", subcore_axis_name="s")
    k = pl.kernel(
        _sc2_body,
        out_type=jax.ShapeDtypeStruct((2 * T * NPAD, HID), _f32),
        mesh=mesh,
        scratch_types=[
            pltpu.VMEM((CH2,), _i32),
            pltpu.VMEM((CH2,), _i32),
            pltpu.VMEM((CH2,), _i32),
            pltpu.VMEM((CH2, HID), _f32),
            pltpu.VMEM((CH2, 16), _f32),
            pltpu.VMEM((CH2, 16), _f32),
            pltpu.VMEM((CH2, 16), _f32),
            pltpu.VMEM((CH2, 16), _f32),
            pltpu.VMEM((T, 16), _f32),
            pltpu.VMEM_SHARED((NPAD, HID), _f32),
            pltpu.SemaphoreType.DMA,
        ],
        compiler_params=pltpu.CompilerParams(use_tc_tiling_on_sc=False),
    )
    return k(ei_flat, ssrc16, sdst16, cmax, hh_flat, dA, dB)


def _layer_norm(m, s, b):
    mu = jnp.mean(m, axis=-1, keepdims=True)
    var = jnp.mean((m - mu) ** 2, axis=-1, keepdims=True)
    return (m - mu) * lax.rsqrt(var + 1e-5) * s + b


def _elu(y):
    return jnp.where(y > 0, y, jnp.exp(y) - 1.0)


def _tcc_body(msg_ref, h0_ref, pe_ref, ln1_s_ref, ln1_b_ref,
              wqT_ref, bq_ref, wkT_ref, bk_ref, wvT_ref, bv_ref,
              eh_ref, eexp_ref, woutT_ref, bout_ref,
              ln2_s_ref, ln2_b_ref, wcT_ref, bc_ref, out_ref):
    seqs = []
    for t in range(T):
        m = msg_ref[0, t] + msg_ref[1, t] + h0_ref[t]
        y = _layer_norm(m, ln1_s_ref[0], ln1_b_ref[0])
        seqs.append(_elu(y) + pe_ref[t])
    x3 = seqs[T - 1]
    q3 = jnp.dot(x3, wqT_ref[...], preferred_element_type=_f32) + bq_ref[0]
    aw = []
    vs = []
    for t in range(T):
        kt = jnp.dot(seqs[t], wkT_ref[...],
                     preferred_element_type=_f32) + bk_ref[0]
        vs.append(jnp.dot(seqs[t], wvT_ref[...],
                          preferred_element_type=_f32) + bv_ref[0])
        aw.append(jnp.dot(q3 * kt, eh_ref[...],
                          preferred_element_type=_f32))
    mx = jnp.maximum(jnp.maximum(aw[0], aw[1]), jnp.maximum(aw[2], aw[3]))
    es = [jnp.exp(a - mx) for a in aw]
    den = es[0] + es[1] + es[2] + es[3]
    ao = None
    for t in range(T):
        w = es[t] / den
        wex = jnp.dot(w, eexp_ref[...], preferred_element_type=_f32)
        ao = wex * vs[t] if ao is None else ao + wex * vs[t]
    out = jnp.dot(ao, woutT_ref[...], preferred_element_type=_f32) + bout_ref[0]
    y2 = _layer_norm(x3 + out, ln2_s_ref[0], ln2_b_ref[0])
    z = _elu(y2)
    out_ref[...] = jnp.dot(z, wcT_ref[...],
                           preferred_element_type=_f32) + bc_ref[0]


def _run_tcc(msg, h0, pe, ln1_s2, ln1_b2, wqT, bq2, wkT, bk2, wvT, bv2,
             eh, eexp, woutT, bout2, ln2_s2, ln2_b2, wcT_pad, bc2_pad):
    full = lambda s: pl.BlockSpec(s, lambda nb: tuple(0 for _ in s))
    return pl.pallas_call(
        _tcc_body,
        grid=(NB,),
        in_specs=[
            pl.BlockSpec((2, T, BLK, HID), lambda nb: (0, 0, nb, 0)),
            pl.BlockSpec((T, BLK, HID), lambda nb: (0, nb, 0)),
            full((T, HID)),
            full((1, HID)), full((1, HID)),
            full((HID, HID)), full((1, HID)),
            full((HID, HID)), full((1, HID)),
            full((HID, HID)), full((1, HID)),
            full((HID, 16)), full((16, HID)),
            full((HID, HID)), full((1, HID)),
            full((1, HID)), full((1, HID)),
            full((HID, HID)), full((1, HID)),
        ],
        out_specs=pl.BlockSpec((BLK, HID), lambda nb: (nb, 0)),
        out_shape=jax.ShapeDtypeStruct((N, HID), _f32),
    )(msg, h0, pe, ln1_s2, ln1_b2, wqT, bq2, wkT, bk2, wvT, bv2,
      eh, eexp, woutT, bout2, ln2_s2, ln2_b2, wcT_pad, bc2_pad)


def _pos_enc():
    pos = jnp.arange(T, dtype=_f32)[:, None]
    div = jnp.exp(jnp.arange(0, HID, 2, dtype=_f32)
                  * (-math.log(10000.0) / HID))
    pe = jnp.zeros((T, HID), dtype=_f32)
    pe = pe.at[:, 0::2].set(jnp.sin(pos * div))
    pe = pe.at[:, 1::2].set(jnp.cos(pos * div))
    return pe


def kernel(x, edge_index, W_in, b_in, W_gat, a_gat, ln1_s, ln1_b, Wqkv, bqkv,
           Wout, bout, ln2_s, ln2_b, Wc, bc):
    winT = W_in.T
    wgatT = W_gat.T
    a1 = a_gat[0, :DH]
    a2 = a_gat[0, DH:]
    sel1 = jnp.kron(jnp.eye(H, dtype=_f32), a1[:, None])
    sel2 = jnp.kron(jnp.eye(H, dtype=_f32), a2[:, None])
    wsrc = wgatT @ jnp.concatenate([sel1, sel1], axis=1)
    wdst = wgatT @ jnp.concatenate([sel2, sel2], axis=1)

    ei = edge_index.astype(_i32)
    ei = jnp.pad(ei, ((0, 0), (0, 0), (0, EPAD - E)))
    ei_flat = ei.reshape(T * 2 * EPAD)

    h0, hh, ss, sd = _run_tca(x, winT, b_in[None, :], wgatT, wsrc, wdst)
    ssrc16 = ss.reshape(T * N, 16)
    sdst16 = sd.reshape(T * N, 16)
    hh_flat = hh.reshape(T * N, HID)

    cm = jnp.max(ss, axis=1) + jnp.max(sd, axis=1)
    cmax = jnp.where(cm > 0, cm, 0.2 * cm)

    den = _run_sc1(ei_flat, ssrc16, sdst16, cmax)
    dA = den[:TNPAD]
    dB = den[TNPAD:]

    msg = _run_sc2(ei_flat, ssrc16, sdst16, cmax, hh_flat, dA, dB)
    msg = msg.reshape(2, T, NPAD, HID)[:, :, :N]

    qs, ks_, vs_ = [], [], []
    for h in range(H):
        qs.append(Wqkv[h * 3 * DH: h * 3 * DH + DH])
        ks_.append(Wqkv[h * 3 * DH + DH: h * 3 * DH + 2 * DH])
        vs_.append(Wqkv[h * 3 * DH + 2 * DH: h * 3 * DH + 3 * DH])
    wq = jnp.concatenate(qs, axis=0)
    wk = jnp.concatenate(ks_, axis=0)
    wv = jnp.concatenate(vs_, axis=0)
    bqkv3 = bqkv.reshape(H, 3 * DH)
    bq = bqkv3[:, :DH].reshape(HID)
    bk = bqkv3[:, DH:2 * DH].reshape(HID)
    bv = bqkv3[:, 2 * DH:].reshape(HID)

    eh = jnp.kron(jnp.eye(H, dtype=_f32), jnp.ones((DH, 1), _f32))
    eh16 = jnp.concatenate([eh, eh], axis=1) / math.sqrt(DH)
    eexp = jnp.concatenate([eh, eh], axis=1).T * 0.5

    wcT_pad = jnp.zeros((HID, HID), _f32).at[:, :NCLS].set(Wc.T)
    bc_pad = jnp.zeros((HID,), _f32).at[:NCLS].set(bc)

    logits = _run_tcc(msg, h0, _pos_enc(), ln1_s[None, :], ln1_b[None, :],
                      wq.T, bq[None, :], wk.T, bk[None, :], wv.T, bv[None, :],
                      eh16, eexp, Wout.T, bout[None, :],
                      ln2_s[None, :], ln2_b[None, :], wcT_pad, bc_pad[None, :])
    return logits[:, :NCLS]


# speedup vs baseline: 39.1292x; 1.0661x over previous
import functools
import math

import jax
import jax.numpy as jnp
from jax import lax
from jax.experimental import pallas as pl
from jax.experimental.pallas import tpu as pltpu
from jax.experimental.pallas import tpu_sc as plsc

N = 10000
E = 160000
T = 4
IN_DIM = 128
HID = 128
H = 8
DH = 16
NCLS = 40

NWORK = 32
EPAD = 163840
EW = EPAD // NWORK
CH1 = 1280
CH2 = 256
BLK = 1000
NB = N // BLK
RPT1 = 2504
TNPAD = 16 * RPT1
RPT2 = 632
NPAD = 16 * RPT2

_f32 = jnp.float32
_i32 = jnp.int32
_GATHER_DNUMS = lax.GatherDimensionNumbers(
    offset_dims=(), collapsed_slice_dims=(0,), start_index_map=(0,))


def _tca_body(x_ref, winT_ref, b_in_ref, wgatT_ref, wsrc_ref, wdst_ref,
              h0_ref, hh_ref, ss_ref, sd_ref):
    xb = x_ref[0]
    h0 = jnp.dot(xb, winT_ref[...], preferred_element_type=_f32) + b_in_ref[0]
    hh = jnp.dot(h0, wgatT_ref[...], preferred_element_type=_f32)
    ss = jnp.dot(h0, wsrc_ref[...], preferred_element_type=_f32)
    sd = jnp.dot(h0, wdst_ref[...], preferred_element_type=_f32)
    h0_ref[0] = h0
    hh_ref[0] = hh
    ss_ref[0] = ss
    sd_ref[0] = sd


def _run_tca(x, winT, b_in2, wgatT, wsrc, wdst):
    full = lambda s: pl.BlockSpec(s, lambda t, nb: tuple(0 for _ in s))
    return pl.pallas_call(
        _tca_body,
        grid=(T, NB),
        in_specs=[
            pl.BlockSpec((1, BLK, IN_DIM), lambda t, nb: (t, nb, 0)),
            full((IN_DIM, HID)),
            full((1, HID)),
            full((HID, HID)),
            full((HID, 16)),
            full((HID, 16)),
        ],
        out_specs=[
            pl.BlockSpec((1, BLK, HID), lambda t, nb: (t, nb, 0)),
            pl.BlockSpec((1, BLK, HID), lambda t, nb: (t, nb, 0)),
            pl.BlockSpec((1, BLK, 16), lambda t, nb: (t, nb, 0)),
            pl.BlockSpec((1, BLK, 16), lambda t, nb: (t, nb, 0)),
        ],
        out_shape=[
            jax.ShapeDtypeStruct((T, N, HID), _f32),
            jax.ShapeDtypeStruct((T, N, HID), _f32),
            jax.ShapeDtypeStruct((T, N, 16), _f32),
            jax.ShapeDtypeStruct((T, N, 16), _f32),
        ],
    )(x, winT, b_in2, wgatT, wsrc, wdst)


def _sc1_body(ei_ref, ssrc_ref, sdst_ref, c_ref, den_ref,
              src_off, dst_off, srow, drow, exb, cbuf, denom_sp, sem):
    cid = lax.axis_index("c")
    sid = lax.axis_index("s")
    wid = sid * 2 + cid
    pltpu.sync_copy(c_ref, cbuf)

    zv = jnp.zeros((16,), _f32)

    def zbuf(g, _):
        exb[g] = zv
        return 0

    lax.fori_loop(0, CH1, zbuf, 0)
    pltpu.sync_copy(exb, denom_sp.at[pl.ds(sid * RPT1, CH1)])
    pltpu.sync_copy(exb.at[pl.ds(0, RPT1 - CH1)],
                    denom_sp.at[pl.ds(sid * RPT1 + CH1, RPT1 - CH1)])
    plsc.subcore_barrier()

    for t in range(T):
        cvec = cbuf[t]

        def chunk(j, _, t=t, cvec=cvec):
            base = wid * EW + j * CH1
            pltpu.sync_copy(ei_ref.at[pl.ds((t * 2) * EPAD + base, CH1)],
                            src_off)
            pltpu.sync_copy(ei_ref.at[pl.ds((t * 2 + 1) * EPAD + base, CH1)],
                            dst_off)

            def addoff(g, _):
                src_off[pl.ds(g * 16, 16)] = src_off[pl.ds(g * 16, 16)] + t * N
                dst_off[pl.ds(g * 16, 16)] = dst_off[pl.ds(g * 16, 16)] + t * N
                return 0

            lax.fori_loop(0, CH1 // 16, addoff, 0)
            cp1 = pltpu.async_copy(ssrc_ref.at[src_off], srow, sem)
            cp2 = pltpu.async_copy(sdst_ref.at[dst_off], drow, sem)
            cp1.wait()
            cp2.wait()

            def body(e, _):
                s = srow[e] + drow[e]
                s = jnp.where(s > 0, s, 0.2 * s) - cvec
                ex = jnp.exp(s)
                valid = jnp.where(base + e < E, 1.0, 0.0).astype(_f32)
                exb[e] = ex * valid
                return 0

            lax.fori_loop(0, CH1, body, 0)
            pltpu.sync_copy(exb, denom_sp.at[dst_off], add=True)
            return 0

        lax.fori_loop(0, EW // CH1, chunk, 0)

    plsc.subcore_barrier()
    pltpu.sync_copy(
        denom_sp.at[pl.ds(sid * RPT1, RPT1)],
        den_ref.at[pl.ds(cid * TNPAD + sid * RPT1, RPT1)])


def _run_sc1(ei_flat, ssrc16, sdst16, cmax):
    mesh = plsc.VectorSubcoreMesh(core_axis_name="c", subcore_axis_name="s")
    k = pl.kernel(
        _sc1_body,
        out_type=jax.ShapeDtypeStruct((2 * TNPAD, 16), _f32),
        mesh=mesh,
        scratch_types=[
            pltpu.VMEM((CH1,), _i32),
            pltpu.VMEM((CH1,), _i32),
            pltpu.VMEM((CH1, 16), _f32),
            pltpu.VMEM((CH1, 16), _f32),
            pltpu.VMEM((CH1, 16), _f32),
            pltpu.VMEM((T, 16), _f32),
            pltpu.VMEM_SHARED((TNPAD, 16), _f32),
            pltpu.SemaphoreType.DMA,
        ],
        compiler_params=pltpu.CompilerParams(use_tc_tiling_on_sc=False),
    )
    return k(ei_flat, ssrc16, sdst16, cmax)


def _densum_body(den_ref, out_ref):
    out_ref[...] = den_ref[0] + den_ref[1] + 1e-16


def _run_densum(den2):
    return pl.pallas_call(
        _densum_body,
        grid=(16,),
        in_specs=[pl.BlockSpec((2, RPT1, 16), lambda nb: (0, nb, 0))],
        out_specs=pl.BlockSpec((RPT1, 16), lambda nb: (nb, 0)),
        out_shape=jax.ShapeDtypeStruct((TNPAD, 16), _f32),
    )(den2)


def _sc2_body(ei_ref, ssrc_ref, sdst_ref, c_ref, hh_ref, dA_ref,
              msg_ref, src_off, dst_loc, dst_off, hhb, srow, drow, dab,
              cbuf, out_sp, sem):
    cid = lax.axis_index("c")
    sid = lax.axis_index("s")
    wid = sid * 2 + cid
    pltpu.sync_copy(c_ref, cbuf)
    zv = jnp.zeros((16,), _f32)
    splats = [jnp.full((16,), h, _i32) for h in range(H)]

    for t in range(T):
        cvec = cbuf[t]

        def zbuf(g, _):
            for kk in range(H):
                hhb[g, pl.ds(kk * 16, 16)] = zv
            return 0

        lax.fori_loop(0, CH2, zbuf, 0)
        for z in range(RPT2 // CH2):
            pltpu.sync_copy(hhb, out_sp.at[pl.ds(sid * RPT2 + z * CH2, CH2)])
        zrem = RPT2 - (RPT2 // CH2) * CH2
        if zrem:
            pltpu.sync_copy(
                hhb.at[pl.ds(0, zrem)],
                out_sp.at[pl.ds(sid * RPT2 + (RPT2 // CH2) * CH2, zrem)])
        plsc.subcore_barrier()

        def chunk(j, _, t=t, cvec=cvec):
            base = wid * EW + j * CH2
            pltpu.sync_copy(ei_ref.at[pl.ds((t * 2) * EPAD + base, CH2)],
                            src_off)
            pltpu.sync_copy(ei_ref.at[pl.ds((t * 2 + 1) * EPAD + base, CH2)],
                            dst_loc)

            def addoff(g, _):
                src_off[pl.ds(g * 16, 16)] = src_off[pl.ds(g * 16, 16)] + t * N
                dst_off[pl.ds(g * 16, 16)] = dst_loc[pl.ds(g * 16, 16)] + t * N
                return 0

            lax.fori_loop(0, CH2 // 16, addoff, 0)
            cps = [
                pltpu.async_copy(hh_ref.at[src_off], hhb, sem),
                pltpu.async_copy(ssrc_ref.at[src_off], srow, sem),
                pltpu.async_copy(sdst_ref.at[dst_off], drow, sem),
                pltpu.async_copy(dA_ref.at[dst_off], dab, sem),
            ]
            for cp in cps:
                cp.wait()

            def body(e, _):
                s = srow[e] + drow[e]
                s = jnp.where(s > 0, s, 0.2 * s) - cvec
                ex = jnp.exp(s)
                valid = jnp.where(base + e < E, 1.0, 0.0).astype(_f32)
                al = ex / dab[e] * valid
                for h in range(H):
                    av = lax.gather(
                        al, splats[h][:, None], _GATHER_DNUMS, (1,),
                        mode=lax.GatherScatterMode.PROMISE_IN_BOUNDS)
                    hv = hhb[e, pl.ds(h * 16, 16)]
                    hhb[e, pl.ds(h * 16, 16)] = hv * av
                return 0

            lax.fori_loop(0, CH2, body, 0)
            pltpu.sync_copy(hhb, out_sp.at[dst_loc], add=True)
            return 0

        lax.fori_loop(0, EW // CH2, chunk, 0)
        plsc.subcore_barrier()
        pltpu.sync_copy(
            out_sp.at[pl.ds(sid * RPT2, RPT2)],
            msg_ref.at[pl.ds(cid * (T * NPAD) + t * NPAD + sid * RPT2,
                             RPT2)])
        plsc.subcore_barrier()


def _run_sc2(ei_flat, ssrc16, sdst16, cmax, hh_flat, dA):
    mesh = plsc.VectorSubcoreMesh(core_axis_name="c", subcore_axis_name="s")
    k = pl.kernel(
        _sc2_body,
        out_type=jax.ShapeDtypeStruct((2 * T * NPAD, HID), _f32),
        mesh=mesh,
        scratch_types=[
            pltpu.VMEM((CH2,), _i32),
            pltpu.VMEM((CH2,), _i32),
            pltpu.VMEM((CH2,), _i32),
            pltpu.VMEM((CH2, HID), _f32),
            pltpu.VMEM((CH2, 16), _f32),
            pltpu.VMEM((CH2, 16), _f32),
            pltpu.VMEM((CH2, 16), _f32),
            pltpu.VMEM((T, 16), _f32),
            pltpu.VMEM_SHARED((NPAD, HID), _f32),
            pltpu.SemaphoreType.DMA,
        ],
        compiler_params=pltpu.CompilerParams(use_tc_tiling_on_sc=False),
    )
    return k(ei_flat, ssrc16, sdst16, cmax, hh_flat, dA)


def _layer_norm(m, s, b):
    mu = jnp.mean(m, axis=-1, keepdims=True)
    var = jnp.mean((m - mu) ** 2, axis=-1, keepdims=True)
    return (m - mu) * lax.rsqrt(var + 1e-5) * s + b


def _elu(y):
    return jnp.where(y > 0, y, jnp.exp(y) - 1.0)


def _tcc_body(msg_ref, h0_ref, pe_ref, ln1_s_ref, ln1_b_ref,
              wqT_ref, bq_ref, wkT_ref, bk_ref, wvT_ref, bv_ref,
              eh_ref, eexp_ref, woutT_ref, bout_ref,
              ln2_s_ref, ln2_b_ref, wcT_ref, bc_ref, out_ref):
    seqs = []
    for t in range(T):
        m = msg_ref[0, t] + msg_ref[1, t] + h0_ref[t]
        y = _layer_norm(m, ln1_s_ref[0], ln1_b_ref[0])
        seqs.append(_elu(y) + pe_ref[t])
    x3 = seqs[T - 1]
    q3 = jnp.dot(x3, wqT_ref[...], preferred_element_type=_f32) + bq_ref[0]
    aw = []
    vs = []
    for t in range(T):
        kt = jnp.dot(seqs[t], wkT_ref[...],
                     preferred_element_type=_f32) + bk_ref[0]
        vs.append(jnp.dot(seqs[t], wvT_ref[...],
                          preferred_element_type=_f32) + bv_ref[0])
        aw.append(jnp.dot(q3 * kt, eh_ref[...],
                          preferred_element_type=_f32))
    mx = jnp.maximum(jnp.maximum(aw[0], aw[1]), jnp.maximum(aw[2], aw[3]))
    es = [jnp.exp(a - mx) for a in aw]
    den = es[0] + es[1] + es[2] + es[3]
    ao = None
    for t in range(T):
        w = es[t] / den
        wex = jnp.dot(w, eexp_ref[...], preferred_element_type=_f32)
        ao = wex * vs[t] if ao is None else ao + wex * vs[t]
    out = jnp.dot(ao, woutT_ref[...], preferred_element_type=_f32) + bout_ref[0]
    y2 = _layer_norm(x3 + out, ln2_s_ref[0], ln2_b_ref[0])
    z = _elu(y2)
    out_ref[...] = jnp.dot(z, wcT_ref[...],
                           preferred_element_type=_f32) + bc_ref[0]


def _run_tcc(msg, h0, pe, ln1_s2, ln1_b2, wqT, bq2, wkT, bk2, wvT, bv2,
             eh, eexp, woutT, bout2, ln2_s2, ln2_b2, wcT_pad, bc2_pad):
    full = lambda s: pl.BlockSpec(s, lambda nb: tuple(0 for _ in s))
    return pl.pallas_call(
        _tcc_body,
        grid=(NB,),
        in_specs=[
            pl.BlockSpec((2, T, BLK, HID), lambda nb: (0, 0, nb, 0)),
            pl.BlockSpec((T, BLK, HID), lambda nb: (0, nb, 0)),
            full((T, HID)),
            full((1, HID)), full((1, HID)),
            full((HID, HID)), full((1, HID)),
            full((HID, HID)), full((1, HID)),
            full((HID, HID)), full((1, HID)),
            full((HID, 16)), full((16, HID)),
            full((HID, HID)), full((1, HID)),
            full((1, HID)), full((1, HID)),
            full((HID, HID)), full((1, HID)),
        ],
        out_specs=pl.BlockSpec((BLK, HID), lambda nb: (nb, 0)),
        out_shape=jax.ShapeDtypeStruct((N, HID), _f32),
    )(msg, h0, pe, ln1_s2, ln1_b2, wqT, bq2, wkT, bk2, wvT, bv2,
      eh, eexp, woutT, bout2, ln2_s2, ln2_b2, wcT_pad, bc2_pad)


def _pos_enc():
    pos = jnp.arange(T, dtype=_f32)[:, None]
    div = jnp.exp(jnp.arange(0, HID, 2, dtype=_f32)
                  * (-math.log(10000.0) / HID))
    pe = jnp.zeros((T, HID), dtype=_f32)
    pe = pe.at[:, 0::2].set(jnp.sin(pos * div))
    pe = pe.at[:, 1::2].set(jnp.cos(pos * div))
    return pe


def kernel(x, edge_index, W_in, b_in, W_gat, a_gat, ln1_s, ln1_b, Wqkv, bqkv,
           Wout, bout, ln2_s, ln2_b, Wc, bc):
    winT = W_in.T
    wgatT = W_gat.T
    a1 = a_gat[0, :DH]
    a2 = a_gat[0, DH:]
    sel1 = jnp.kron(jnp.eye(H, dtype=_f32), a1[:, None])
    sel2 = jnp.kron(jnp.eye(H, dtype=_f32), a2[:, None])
    wsrc = wgatT @ jnp.concatenate([sel1, sel1], axis=1)
    wdst = wgatT @ jnp.concatenate([sel2, sel2], axis=1)

    ei = edge_index.astype(_i32)
    ei = jnp.pad(ei, ((0, 0), (0, 0), (0, EPAD - E)))
    ei_flat = ei.reshape(T * 2 * EPAD)

    h0, hh, ss, sd = _run_tca(x, winT, b_in[None, :], wgatT, wsrc, wdst)
    ssrc16 = ss.reshape(T * N, 16)
    sdst16 = sd.reshape(T * N, 16)
    hh_flat = hh.reshape(T * N, HID)

    cm = jnp.max(ss, axis=1) + jnp.max(sd, axis=1)
    cmax = jnp.where(cm > 0, cm, 0.2 * cm)

    den = _run_sc1(ei_flat, ssrc16, sdst16, cmax)
    dsum = _run_densum(den.reshape(2, TNPAD, 16))

    msg = _run_sc2(ei_flat, ssrc16, sdst16, cmax, hh_flat, dsum)
    msg = msg.reshape(2, T, NPAD, HID)[:, :, :N]

    qs, ks_, vs_ = [], [], []
    for h in range(H):
        qs.append(Wqkv[h * 3 * DH: h * 3 * DH + DH])
        ks_.append(Wqkv[h * 3 * DH + DH: h * 3 * DH + 2 * DH])
        vs_.append(Wqkv[h * 3 * DH + 2 * DH: h * 3 * DH + 3 * DH])
    wq = jnp.concatenate(qs, axis=0)
    wk = jnp.concatenate(ks_, axis=0)
    wv = jnp.concatenate(vs_, axis=0)
    bqkv3 = bqkv.reshape(H, 3 * DH)
    bq = bqkv3[:, :DH].reshape(HID)
    bk = bqkv3[:, DH:2 * DH].reshape(HID)
    bv = bqkv3[:, 2 * DH:].reshape(HID)

    eh = jnp.kron(jnp.eye(H, dtype=_f32), jnp.ones((DH, 1), _f32))
    eh16 = jnp.concatenate([eh, eh], axis=1) / math.sqrt(DH)
    eexp = jnp.concatenate([eh, eh], axis=1).T * 0.5

    wcT_pad = jnp.zeros((HID, HID), _f32).at[:, :NCLS].set(Wc.T)
    bc_pad = jnp.zeros((HID,), _f32).at[:NCLS].set(bc)

    logits = _run_tcc(msg, h0, _pos_enc(), ln1_s[None, :], ln1_b[None, :],
                      wq.T, bq[None, :], wk.T, bk[None, :], wv.T, bv[None, :],
                      eh16, eexp, Wout.T, bout[None, :],
                      ln2_s[None, :], ln2_b[None, :], wcT_pad, bc_pad[None, :])
    return logits[:, :NCLS]


# speedup vs baseline: 39.7221x; 1.0152x over previous
import functools
import math

import jax
import jax.numpy as jnp
from jax import lax
from jax.experimental import pallas as pl
from jax.experimental.pallas import tpu as pltpu
from jax.experimental.pallas import tpu_sc as plsc

N = 10000
E = 160000
T = 4
IN_DIM = 128
HID = 128
H = 8
DH = 16
NCLS = 40

NWORK = 32
EPAD = 163840
EW = EPAD // NWORK
CH1 = 1280
CH2 = 256
BLK = 1000
NB = N // BLK
RPT1 = 2504
TNPAD = 16 * RPT1
RPT2 = 632
NPAD = 16 * RPT2

_f32 = jnp.float32
_i32 = jnp.int32
_GATHER_DNUMS = lax.GatherDimensionNumbers(
    offset_dims=(), collapsed_slice_dims=(0,), start_index_map=(0,))


def _tca_body(x_ref, winT_ref, b_in_ref, wgatT_ref, wsrc_ref, wdst_ref,
              h0_ref, hh_ref, ss_ref, sd_ref):
    xb = x_ref[0]
    h0 = jnp.dot(xb, winT_ref[...], preferred_element_type=_f32) + b_in_ref[0]
    hh = jnp.dot(h0, wgatT_ref[...], preferred_element_type=_f32)
    ss = jnp.dot(h0, wsrc_ref[...], preferred_element_type=_f32)
    sd = jnp.dot(h0, wdst_ref[...], preferred_element_type=_f32)
    h0_ref[0] = h0
    hh_ref[0] = hh
    ss_ref[0] = ss
    sd_ref[0] = sd


def _run_tca(x, winT, b_in2, wgatT, wsrc, wdst):
    full = lambda s: pl.BlockSpec(s, lambda t, nb: tuple(0 for _ in s))
    return pl.pallas_call(
        _tca_body,
        grid=(T, NB),
        in_specs=[
            pl.BlockSpec((1, BLK, IN_DIM), lambda t, nb: (t, nb, 0)),
            full((IN_DIM, HID)),
            full((1, HID)),
            full((HID, HID)),
            full((HID, 16)),
            full((HID, 16)),
        ],
        out_specs=[
            pl.BlockSpec((1, BLK, HID), lambda t, nb: (t, nb, 0)),
            pl.BlockSpec((1, BLK, HID), lambda t, nb: (t, nb, 0)),
            pl.BlockSpec((1, BLK, 16), lambda t, nb: (t, nb, 0)),
            pl.BlockSpec((1, BLK, 16), lambda t, nb: (t, nb, 0)),
        ],
        out_shape=[
            jax.ShapeDtypeStruct((T, N, HID), _f32),
            jax.ShapeDtypeStruct((T, N, HID), _f32),
            jax.ShapeDtypeStruct((T, N, 16), _f32),
            jax.ShapeDtypeStruct((T, N, 16), _f32),
        ],
    )(x, winT, b_in2, wgatT, wsrc, wdst)


def _sc1_body(ei_ref, ssrc_ref, sdst_ref, c_ref, den_ref, ex_ref,
              src_off, dst_off, srow, drow, exb, cbuf, denom_sp, sem):
    cid = lax.axis_index("c")
    sid = lax.axis_index("s")
    wid = sid * 2 + cid
    pltpu.sync_copy(c_ref, cbuf)

    zv = jnp.zeros((16,), _f32)

    def zbuf(g, _):
        exb[g] = zv
        return 0

    lax.fori_loop(0, CH1, zbuf, 0)
    pltpu.sync_copy(exb, denom_sp.at[pl.ds(sid * RPT1, CH1)])
    pltpu.sync_copy(exb.at[pl.ds(0, RPT1 - CH1)],
                    denom_sp.at[pl.ds(sid * RPT1 + CH1, RPT1 - CH1)])
    plsc.subcore_barrier()

    for t in range(T):
        cvec = cbuf[t]

        def chunk(j, _, t=t, cvec=cvec):
            base = wid * EW + j * CH1
            pltpu.sync_copy(ei_ref.at[pl.ds((t * 2) * EPAD + base, CH1)],
                            src_off)
            pltpu.sync_copy(ei_ref.at[pl.ds((t * 2 + 1) * EPAD + base, CH1)],
                            dst_off)

            def addoff(g, _):
                src_off[pl.ds(g * 16, 16)] = src_off[pl.ds(g * 16, 16)] + t * N
                dst_off[pl.ds(g * 16, 16)] = dst_off[pl.ds(g * 16, 16)] + t * N
                return 0

            lax.fori_loop(0, CH1 // 16, addoff, 0)
            cp1 = pltpu.async_copy(ssrc_ref.at[src_off], srow, sem)
            cp2 = pltpu.async_copy(sdst_ref.at[dst_off], drow, sem)
            cp1.wait()
            cp2.wait()

            def body(e, _):
                s = srow[e] + drow[e]
                s = jnp.where(s > 0, s, 0.2 * s) - cvec
                ex = jnp.exp(s)
                valid = jnp.where(base + e < E, 1.0, 0.0).astype(_f32)
                exb[e] = ex * valid
                return 0

            lax.fori_loop(0, CH1, body, 0)
            pltpu.sync_copy(exb, ex_ref.at[pl.ds(t * EPAD + base, CH1)])
            pltpu.sync_copy(exb, denom_sp.at[dst_off], add=True)
            return 0

        lax.fori_loop(0, EW // CH1, chunk, 0)

    plsc.subcore_barrier()
    pltpu.sync_copy(
        denom_sp.at[pl.ds(sid * RPT1, RPT1)],
        den_ref.at[pl.ds(cid * TNPAD + sid * RPT1, RPT1)])


def _run_sc1(ei_flat, ssrc16, sdst16, cmax):
    mesh = plsc.VectorSubcoreMesh(core_axis_name="c", subcore_axis_name="s")
    k = pl.kernel(
        _sc1_body,
        out_type=[jax.ShapeDtypeStruct((2 * TNPAD, 16), _f32),
                  jax.ShapeDtypeStruct((T * EPAD, 16), _f32)],
        mesh=mesh,
        scratch_types=[
            pltpu.VMEM((CH1,), _i32),
            pltpu.VMEM((CH1,), _i32),
            pltpu.VMEM((CH1, 16), _f32),
            pltpu.VMEM((CH1, 16), _f32),
            pltpu.VMEM((CH1, 16), _f32),
            pltpu.VMEM((T, 16), _f32),
            pltpu.VMEM_SHARED((TNPAD, 16), _f32),
            pltpu.SemaphoreType.DMA,
        ],
        compiler_params=pltpu.CompilerParams(use_tc_tiling_on_sc=False),
    )
    return k(ei_flat, ssrc16, sdst16, cmax)


def _densum_body(den_ref, out_ref):
    out_ref[...] = den_ref[0] + den_ref[1] + 1e-16


def _run_densum(den2):
    return pl.pallas_call(
        _densum_body,
        grid=(16,),
        in_specs=[pl.BlockSpec((2, RPT1, 16), lambda nb: (0, nb, 0))],
        out_specs=pl.BlockSpec((RPT1, 16), lambda nb: (nb, 0)),
        out_shape=jax.ShapeDtypeStruct((TNPAD, 16), _f32),
    )(den2)


def _sc2_body(ei_ref, ex_ref, hh_ref, dA_ref,
              msg_ref, src_off, dst_loc, dst_off, hhb, exb, dab,
              out_sp, sem):
    cid = lax.axis_index("c")
    sid = lax.axis_index("s")
    wid = sid * 2 + cid
    zv = jnp.zeros((16,), _f32)
    splats = [jnp.full((16,), h, _i32) for h in range(H)]

    for t in range(T):
        def zbuf(g, _):
            for kk in range(H):
                hhb[g, pl.ds(kk * 16, 16)] = zv
            return 0

        lax.fori_loop(0, CH2, zbuf, 0)
        for z in range(RPT2 // CH2):
            pltpu.sync_copy(hhb, out_sp.at[pl.ds(sid * RPT2 + z * CH2, CH2)])
        zrem = RPT2 - (RPT2 // CH2) * CH2
        if zrem:
            pltpu.sync_copy(
                hhb.at[pl.ds(0, zrem)],
                out_sp.at[pl.ds(sid * RPT2 + (RPT2 // CH2) * CH2, zrem)])
        plsc.subcore_barrier()

        def chunk(j, _, t=t):
            base = wid * EW + j * CH2
            pltpu.sync_copy(ei_ref.at[pl.ds((t * 2) * EPAD + base, CH2)],
                            src_off)
            pltpu.sync_copy(ei_ref.at[pl.ds((t * 2 + 1) * EPAD + base, CH2)],
                            dst_loc)

            def addoff(g, _):
                src_off[pl.ds(g * 16, 16)] = src_off[pl.ds(g * 16, 16)] + t * N
                dst_off[pl.ds(g * 16, 16)] = dst_loc[pl.ds(g * 16, 16)] + t * N
                return 0

            lax.fori_loop(0, CH2 // 16, addoff, 0)
            cps = [
                pltpu.async_copy(hh_ref.at[src_off], hhb, sem),
                pltpu.async_copy(ex_ref.at[pl.ds(t * EPAD + base, CH2)],
                                 exb, sem),
                pltpu.async_copy(dA_ref.at[dst_off], dab, sem),
            ]
            for cp in cps:
                cp.wait()

            def body(e, _):
                al = exb[e] / dab[e]
                for h in range(H):
                    av = lax.gather(
                        al, splats[h][:, None], _GATHER_DNUMS, (1,),
                        mode=lax.GatherScatterMode.PROMISE_IN_BOUNDS)
                    hv = hhb[e, pl.ds(h * 16, 16)]
                    hhb[e, pl.ds(h * 16, 16)] = hv * av
                return 0

            lax.fori_loop(0, CH2, body, 0)
            pltpu.sync_copy(hhb, out_sp.at[dst_loc], add=True)
            return 0

        lax.fori_loop(0, EW // CH2, chunk, 0)
        plsc.subcore_barrier()
        pltpu.sync_copy(
            out_sp.at[pl.ds(sid * RPT2, RPT2)],
            msg_ref.at[pl.ds(cid * (T * NPAD) + t * NPAD + sid * RPT2,
                             RPT2)])
        plsc.subcore_barrier()


def _run_sc2(ei_flat, ex, hh_flat, dA):
    mesh = plsc.VectorSubcoreMesh(core_axis_name="c", subcore_axis_name="s")
    k = pl.kernel(
        _sc2_body,
        out_type=jax.ShapeDtypeStruct((2 * T * NPAD, HID), _f32),
        mesh=mesh,
        scratch_types=[
            pltpu.VMEM((CH2,), _i32),
            pltpu.VMEM((CH2,), _i32),
            pltpu.VMEM((CH2,), _i32),
            pltpu.VMEM((CH2, HID), _f32),
            pltpu.VMEM((CH2, 16), _f32),
            pltpu.VMEM((CH2, 16), _f32),
            pltpu.VMEM_SHARED((NPAD, HID), _f32),
            pltpu.SemaphoreType.DMA,
        ],
        compiler_params=pltpu.CompilerParams(use_tc_tiling_on_sc=False),
    )
    return k(ei_flat, ex, hh_flat, dA)


def _layer_norm(m, s, b):
    mu = jnp.mean(m, axis=-1, keepdims=True)
    var = jnp.mean((m - mu) ** 2, axis=-1, keepdims=True)
    return (m - mu) * lax.rsqrt(var + 1e-5) * s + b


def _elu(y):
    return jnp.where(y > 0, y, jnp.exp(y) - 1.0)


def _tcc_body(msg_ref, h0_ref, pe_ref, ln1_s_ref, ln1_b_ref,
              wqT_ref, bq_ref, wkT_ref, bk_ref, wvT_ref, bv_ref,
              eh_ref, eexp_ref, woutT_ref, bout_ref,
              ln2_s_ref, ln2_b_ref, wcT_ref, bc_ref, out_ref):
    seqs = []
    for t in range(T):
        m = msg_ref[0, t] + msg_ref[1, t] + h0_ref[t]
        y = _layer_norm(m, ln1_s_ref[0], ln1_b_ref[0])
        seqs.append(_elu(y) + pe_ref[t])
    x3 = seqs[T - 1]
    q3 = jnp.dot(x3, wqT_ref[...], preferred_element_type=_f32) + bq_ref[0]
    aw = []
    vs = []
    for t in range(T):
        kt = jnp.dot(seqs[t], wkT_ref[...],
                     preferred_element_type=_f32) + bk_ref[0]
        vs.append(jnp.dot(seqs[t], wvT_ref[...],
                          preferred_element_type=_f32) + bv_ref[0])
        aw.append(jnp.dot(q3 * kt, eh_ref[...],
                          preferred_element_type=_f32))
    mx = jnp.maximum(jnp.maximum(aw[0], aw[1]), jnp.maximum(aw[2], aw[3]))
    es = [jnp.exp(a - mx) for a in aw]
    den = es[0] + es[1] + es[2] + es[3]
    ao = None
    for t in range(T):
        w = es[t] / den
        wex = jnp.dot(w, eexp_ref[...], preferred_element_type=_f32)
        ao = wex * vs[t] if ao is None else ao + wex * vs[t]
    out = jnp.dot(ao, woutT_ref[...], preferred_element_type=_f32) + bout_ref[0]
    y2 = _layer_norm(x3 + out, ln2_s_ref[0], ln2_b_ref[0])
    z = _elu(y2)
    out_ref[...] = jnp.dot(z, wcT_ref[...],
                           preferred_element_type=_f32) + bc_ref[0]


def _run_tcc(msg, h0, pe, ln1_s2, ln1_b2, wqT, bq2, wkT, bk2, wvT, bv2,
             eh, eexp, woutT, bout2, ln2_s2, ln2_b2, wcT_pad, bc2_pad):
    full = lambda s: pl.BlockSpec(s, lambda nb: tuple(0 for _ in s))
    return pl.pallas_call(
        _tcc_body,
        grid=(NB,),
        in_specs=[
            pl.BlockSpec((2, T, BLK, HID), lambda nb: (0, 0, nb, 0)),
            pl.BlockSpec((T, BLK, HID), lambda nb: (0, nb, 0)),
            full((T, HID)),
            full((1, HID)), full((1, HID)),
            full((HID, HID)), full((1, HID)),
            full((HID, HID)), full((1, HID)),
            full((HID, HID)), full((1, HID)),
            full((HID, 16)), full((16, HID)),
            full((HID, HID)), full((1, HID)),
            full((1, HID)), full((1, HID)),
            full((HID, HID)), full((1, HID)),
        ],
        out_specs=pl.BlockSpec((BLK, HID), lambda nb: (nb, 0)),
        out_shape=jax.ShapeDtypeStruct((N, HID), _f32),
    )(msg, h0, pe, ln1_s2, ln1_b2, wqT, bq2, wkT, bk2, wvT, bv2,
      eh, eexp, woutT, bout2, ln2_s2, ln2_b2, wcT_pad, bc2_pad)


def _pos_enc():
    pos = jnp.arange(T, dtype=_f32)[:, None]
    div = jnp.exp(jnp.arange(0, HID, 2, dtype=_f32)
                  * (-math.log(10000.0) / HID))
    pe = jnp.zeros((T, HID), dtype=_f32)
    pe = pe.at[:, 0::2].set(jnp.sin(pos * div))
    pe = pe.at[:, 1::2].set(jnp.cos(pos * div))
    return pe


def kernel(x, edge_index, W_in, b_in, W_gat, a_gat, ln1_s, ln1_b, Wqkv, bqkv,
           Wout, bout, ln2_s, ln2_b, Wc, bc):
    winT = W_in.T
    wgatT = W_gat.T
    a1 = a_gat[0, :DH]
    a2 = a_gat[0, DH:]
    sel1 = jnp.kron(jnp.eye(H, dtype=_f32), a1[:, None])
    sel2 = jnp.kron(jnp.eye(H, dtype=_f32), a2[:, None])
    wsrc = wgatT @ jnp.concatenate([sel1, sel1], axis=1)
    wdst = wgatT @ jnp.concatenate([sel2, sel2], axis=1)

    ei = edge_index.astype(_i32)
    ei = jnp.pad(ei, ((0, 0), (0, 0), (0, EPAD - E)))
    ei_flat = ei.reshape(T * 2 * EPAD)

    h0, hh, ss, sd = _run_tca(x, winT, b_in[None, :], wgatT, wsrc, wdst)
    ssrc16 = ss.reshape(T * N, 16)
    sdst16 = sd.reshape(T * N, 16)
    hh_flat = hh.reshape(T * N, HID)

    cm = jnp.max(ss, axis=1) + jnp.max(sd, axis=1)
    cmax = jnp.where(cm > 0, cm, 0.2 * cm)

    den, ex = _run_sc1(ei_flat, ssrc16, sdst16, cmax)
    dsum = _run_densum(den.reshape(2, TNPAD, 16))

    msg = _run_sc2(ei_flat, ex, hh_flat, dsum)
    msg = msg.reshape(2, T, NPAD, HID)[:, :, :N]

    qs, ks_, vs_ = [], [], []
    for h in range(H):
        qs.append(Wqkv[h * 3 * DH: h * 3 * DH + DH])
        ks_.append(Wqkv[h * 3 * DH + DH: h * 3 * DH + 2 * DH])
        vs_.append(Wqkv[h * 3 * DH + 2 * DH: h * 3 * DH + 3 * DH])
    wq = jnp.concatenate(qs, axis=0)
    wk = jnp.concatenate(ks_, axis=0)
    wv = jnp.concatenate(vs_, axis=0)
    bqkv3 = bqkv.reshape(H, 3 * DH)
    bq = bqkv3[:, :DH].reshape(HID)
    bk = bqkv3[:, DH:2 * DH].reshape(HID)
    bv = bqkv3[:, 2 * DH:].reshape(HID)

    eh = jnp.kron(jnp.eye(H, dtype=_f32), jnp.ones((DH, 1), _f32))
    eh16 = jnp.concatenate([eh, eh], axis=1) / math.sqrt(DH)
    eexp = jnp.concatenate([eh, eh], axis=1).T * 0.5

    wcT_pad = jnp.zeros((HID, HID), _f32).at[:, :NCLS].set(Wc.T)
    bc_pad = jnp.zeros((HID,), _f32).at[:NCLS].set(bc)

    logits = _run_tcc(msg, h0, _pos_enc(), ln1_s[None, :], ln1_b[None, :],
                      wq.T, bq[None, :], wk.T, bk[None, :], wv.T, bv[None, :],
                      eh16, eexp, Wout.T, bout[None, :],
                      ln2_s[None, :], ln2_b[None, :], wcT_pad, bc_pad[None, :])
    return logits[:, :NCLS]


# speedup vs baseline: 50.0285x; 1.2595x over previous
import functools
import math

import jax
import jax.numpy as jnp
from jax import lax
from jax.experimental import pallas as pl
from jax.experimental.pallas import tpu as pltpu
from jax.experimental.pallas import tpu_sc as plsc

N = 10000
E = 160000
T = 4
IN_DIM = 128
HID = 128
H = 8
DH = 16
NCLS = 40

NWORK = 32
EPAD = 163840
EW = EPAD // NWORK
CH1 = 1280
CH2 = 64
NBUF2 = 4
BLK = 1000
NB = N // BLK
RPT1 = 2504
TNPAD = 16 * RPT1
RPT2 = 632
NPAD = 16 * RPT2

_f32 = jnp.float32
_i32 = jnp.int32
_GATHER_DNUMS = lax.GatherDimensionNumbers(
    offset_dims=(), collapsed_slice_dims=(0,), start_index_map=(0,))


def _tca_body(x_ref, winT_ref, b_in_ref, wgatT_ref, wsrc_ref, wdst_ref,
              h0_ref, hh_ref, ss_ref, sd_ref):
    xb = x_ref[0]
    h0 = jnp.dot(xb, winT_ref[...], preferred_element_type=_f32) + b_in_ref[0]
    hh = jnp.dot(h0, wgatT_ref[...], preferred_element_type=_f32)
    ss = jnp.dot(h0, wsrc_ref[...], preferred_element_type=_f32)
    sd = jnp.dot(h0, wdst_ref[...], preferred_element_type=_f32)
    h0_ref[0] = h0
    hh_ref[0] = hh
    ss_ref[0] = ss
    sd_ref[0] = sd


def _run_tca(x, winT, b_in2, wgatT, wsrc, wdst):
    full = lambda s: pl.BlockSpec(s, lambda t, nb: tuple(0 for _ in s))
    return pl.pallas_call(
        _tca_body,
        grid=(T, NB),
        in_specs=[
            pl.BlockSpec((1, BLK, IN_DIM), lambda t, nb: (t, nb, 0)),
            full((IN_DIM, HID)),
            full((1, HID)),
            full((HID, HID)),
            full((HID, 16)),
            full((HID, 16)),
        ],
        out_specs=[
            pl.BlockSpec((1, BLK, HID), lambda t, nb: (t, nb, 0)),
            pl.BlockSpec((1, BLK, HID), lambda t, nb: (t, nb, 0)),
            pl.BlockSpec((1, BLK, 16), lambda t, nb: (t, nb, 0)),
            pl.BlockSpec((1, BLK, 16), lambda t, nb: (t, nb, 0)),
        ],
        out_shape=[
            jax.ShapeDtypeStruct((T, N, HID), _f32),
            jax.ShapeDtypeStruct((T, N, HID), _f32),
            jax.ShapeDtypeStruct((T, N, 16), _f32),
            jax.ShapeDtypeStruct((T, N, 16), _f32),
        ],
    )(x, winT, b_in2, wgatT, wsrc, wdst)


def _sc1_body(ei_ref, ssrc_ref, sdst_ref, c_ref, den_ref, ex_ref,
              src_off, dst_off, srow, drow, exb, cbuf, denom_sp, sem):
    cid = lax.axis_index("c")
    sid = lax.axis_index("s")
    wid = sid * 2 + cid
    pltpu.sync_copy(c_ref, cbuf)

    zv = jnp.zeros((16,), _f32)

    def zbuf(g, _):
        exb[g] = zv
        return 0

    lax.fori_loop(0, CH1, zbuf, 0)
    pltpu.sync_copy(exb, denom_sp.at[pl.ds(sid * RPT1, CH1)])
    pltpu.sync_copy(exb.at[pl.ds(0, RPT1 - CH1)],
                    denom_sp.at[pl.ds(sid * RPT1 + CH1, RPT1 - CH1)])
    plsc.subcore_barrier()

    for t in range(T):
        cvec = cbuf[t]

        def chunk(j, _, t=t, cvec=cvec):
            base = wid * EW + j * CH1
            pltpu.sync_copy(ei_ref.at[pl.ds((t * 2) * EPAD + base, CH1)],
                            src_off)
            pltpu.sync_copy(ei_ref.at[pl.ds((t * 2 + 1) * EPAD + base, CH1)],
                            dst_off)

            def addoff(g, _):
                src_off[pl.ds(g * 16, 16)] = src_off[pl.ds(g * 16, 16)] + t * N
                dst_off[pl.ds(g * 16, 16)] = dst_off[pl.ds(g * 16, 16)] + t * N
                return 0

            lax.fori_loop(0, CH1 // 16, addoff, 0)
            cp1 = pltpu.async_copy(ssrc_ref.at[src_off], srow, sem)
            cp2 = pltpu.async_copy(sdst_ref.at[dst_off], drow, sem)
            cp1.wait()
            cp2.wait()

            def body(e, _):
                s = srow[e] + drow[e]
                s = jnp.where(s > 0, s, 0.2 * s) - cvec
                ex = jnp.exp(s)
                valid = jnp.where(base + e < E, 1.0, 0.0).astype(_f32)
                exb[e] = ex * valid
                return 0

            lax.fori_loop(0, CH1, body, 0)
            pltpu.sync_copy(exb, ex_ref.at[pl.ds(t * EPAD + base, CH1)])
            pltpu.sync_copy(exb, denom_sp.at[dst_off], add=True)
            return 0

        lax.fori_loop(0, EW // CH1, chunk, 0)

    plsc.subcore_barrier()
    pltpu.sync_copy(
        denom_sp.at[pl.ds(sid * RPT1, RPT1)],
        den_ref.at[pl.ds(cid * TNPAD + sid * RPT1, RPT1)])


def _run_sc1(ei_flat, ssrc16, sdst16, cmax):
    mesh = plsc.VectorSubcoreMesh(core_axis_name="c", subcore_axis_name="s")
    k = pl.kernel(
        _sc1_body,
        out_type=[jax.ShapeDtypeStruct((2 * TNPAD, 16), _f32),
                  jax.ShapeDtypeStruct((T * EPAD, 16), _f32)],
        mesh=mesh,
        scratch_types=[
            pltpu.VMEM((CH1,), _i32),
            pltpu.VMEM((CH1,), _i32),
            pltpu.VMEM((CH1, 16), _f32),
            pltpu.VMEM((CH1, 16), _f32),
            pltpu.VMEM((CH1, 16), _f32),
            pltpu.VMEM((T, 16), _f32),
            pltpu.VMEM_SHARED((TNPAD, 16), _f32),
            pltpu.SemaphoreType.DMA,
        ],
        compiler_params=pltpu.CompilerParams(use_tc_tiling_on_sc=False),
    )
    return k(ei_flat, ssrc16, sdst16, cmax)


def _densum_body(den_ref, out_ref):
    out_ref[...] = 1.0 / (den_ref[0] + den_ref[1] + 1e-16)


def _run_densum(den2):
    return pl.pallas_call(
        _densum_body,
        grid=(16,),
        in_specs=[pl.BlockSpec((2, RPT1, 16), lambda nb: (0, nb, 0))],
        out_specs=pl.BlockSpec((RPT1, 16), lambda nb: (nb, 0)),
        out_shape=jax.ShapeDtypeStruct((TNPAD, 16), _f32),
    )(den2)


def _sc2_body(ei_ref, ex_ref, hh_ref, dA_ref, msg_ref,
              srco, dstl, dsto, hhb, exb, dab, out_sp,
              gs0, gs1, gs2, gs3, ss0, ss1, ss2, ss3):
    cid = lax.axis_index("c")
    sid = lax.axis_index("s")
    wid = sid * 2 + cid
    zv = jnp.zeros((16,), _f32)
    splats = [jnp.full((16,), h, _i32) for h in range(H)]
    gsems = [gs0, gs1, gs2, gs3]
    ssems = [ss0, ss1, ss2, ss3]
    NCH = EW // CH2
    NP = NCH // NBUF2

    def load_fire(t, j, u):
        base = wid * EW + j * CH2
        pltpu.sync_copy(ei_ref.at[pl.ds((t * 2) * EPAD + base, CH2)],
                        srco.at[u])
        pltpu.sync_copy(ei_ref.at[pl.ds((t * 2 + 1) * EPAD + base, CH2)],
                        dstl.at[u])

        def addoff(g, _):
            srco[u, pl.ds(g * 16, 16)] = srco[u, pl.ds(g * 16, 16)] + t * N
            dsto[u, pl.ds(g * 16, 16)] = dstl[u, pl.ds(g * 16, 16)] + t * N
            return 0

        lax.fori_loop(0, CH2 // 16, addoff, 0)
        pltpu.async_copy(hh_ref.at[srco.at[u]], hhb.at[u], gsems[u])
        pltpu.async_copy(ex_ref.at[pl.ds(t * EPAD + base, CH2)],
                         exb.at[u], gsems[u])
        pltpu.async_copy(dA_ref.at[dsto.at[u]], dab.at[u], gsems[u])

    def wait_gathers(t, j, u):
        base = wid * EW + j * CH2
        pltpu.make_async_copy(hh_ref.at[srco.at[u]], hhb.at[u],
                              gsems[u]).wait()
        pltpu.make_async_copy(ex_ref.at[pl.ds(t * EPAD + base, CH2)],
                              exb.at[u], gsems[u]).wait()
        pltpu.make_async_copy(dA_ref.at[dsto.at[u]], dab.at[u],
                              gsems[u]).wait()

    def wait_scatter(u):
        pltpu.make_async_copy(hhb.at[u], out_sp.at[pl.ds(0, CH2)],
                              ssems[u]).wait()

    def compute(u):
        def body(e, _):
            al = exb[u, e] * dab[u, e]
            for h in range(H):
                av = lax.gather(
                    al, splats[h][:, None], _GATHER_DNUMS, (1,),
                    mode=lax.GatherScatterMode.PROMISE_IN_BOUNDS)
                hv = hhb[u, e, pl.ds(h * 16, 16)]
                hhb[u, e, pl.ds(h * 16, 16)] = hv * av
            return 0

        lax.fori_loop(0, CH2, body, 0)

    for t in range(T):
        def zbuf(g, _):
            for kk in range(H):
                hhb[0, g, pl.ds(kk * 16, 16)] = zv
            return 0

        lax.fori_loop(0, CH2, zbuf, 0)
        for z in range(RPT2 // CH2):
            pltpu.sync_copy(hhb.at[0],
                            out_sp.at[pl.ds(sid * RPT2 + z * CH2, CH2)])
        zrem = RPT2 - (RPT2 // CH2) * CH2
        if zrem:
            pltpu.sync_copy(
                hhb.at[0].at[pl.ds(0, zrem)],
                out_sp.at[pl.ds(sid * RPT2 + (RPT2 // CH2) * CH2, zrem)])
        plsc.subcore_barrier()

        load_fire(t, 0, 0)

        def quad(g, _, t=t):
            for u in range(NBUF2):
                j = g * NBUF2 + u
                nu = (u + 1) % NBUF2
                if u < NBUF2 - 1:
                    @pl.when(g > 0)
                    def _(nu=nu):
                        wait_scatter(nu)

                    load_fire(t, j + 1, nu)
                else:
                    @pl.when(g < NP - 1)
                    def _(j=j):
                        wait_scatter(0)
                        load_fire(t, j + 1, 0)
                wait_gathers(t, j, u)
                compute(u)
                pltpu.async_copy(hhb.at[u], out_sp.at[dstl.at[u]],
                                 ssems[u], add=True)
            return 0

        lax.fori_loop(0, NP, quad, 0)
        for u in range(NBUF2):
            wait_scatter(u)
        plsc.subcore_barrier()
        pltpu.sync_copy(
            out_sp.at[pl.ds(sid * RPT2, RPT2)],
            msg_ref.at[pl.ds(cid * (T * NPAD) + t * NPAD + sid * RPT2,
                             RPT2)])
        plsc.subcore_barrier()


def _run_sc2(ei_flat, ex, hh_flat, dA):
    mesh = plsc.VectorSubcoreMesh(core_axis_name="c", subcore_axis_name="s")
    k = pl.kernel(
        _sc2_body,
        out_type=jax.ShapeDtypeStruct((2 * T * NPAD, HID), _f32),
        mesh=mesh,
        scratch_types=[
            pltpu.VMEM((NBUF2, CH2), _i32),
            pltpu.VMEM((NBUF2, CH2), _i32),
            pltpu.VMEM((NBUF2, CH2), _i32),
            pltpu.VMEM((NBUF2, CH2, HID), _f32),
            pltpu.VMEM((NBUF2, CH2, 16), _f32),
            pltpu.VMEM((NBUF2, CH2, 16), _f32),
            pltpu.VMEM_SHARED((NPAD, HID), _f32),
            pltpu.SemaphoreType.DMA,
            pltpu.SemaphoreType.DMA,
            pltpu.SemaphoreType.DMA,
            pltpu.SemaphoreType.DMA,
            pltpu.SemaphoreType.DMA,
            pltpu.SemaphoreType.DMA,
            pltpu.SemaphoreType.DMA,
            pltpu.SemaphoreType.DMA,
        ],
        compiler_params=pltpu.CompilerParams(use_tc_tiling_on_sc=False),
    )
    return k(ei_flat, ex, hh_flat, dA)


def _layer_norm(m, s, b):
    mu = jnp.mean(m, axis=-1, keepdims=True)
    var = jnp.mean((m - mu) ** 2, axis=-1, keepdims=True)
    return (m - mu) * lax.rsqrt(var + 1e-5) * s + b


def _elu(y):
    return jnp.where(y > 0, y, jnp.exp(y) - 1.0)


def _tcc_body(msg_ref, h0_ref, pe_ref, ln1_s_ref, ln1_b_ref,
              wqT_ref, bq_ref, wkT_ref, bk_ref, wvT_ref, bv_ref,
              eh_ref, eexp_ref, woutT_ref, bout_ref,
              ln2_s_ref, ln2_b_ref, wcT_ref, bc_ref, out_ref):
    seqs = []
    for t in range(T):
        m = msg_ref[0, t] + msg_ref[1, t] + h0_ref[t]
        y = _layer_norm(m, ln1_s_ref[0], ln1_b_ref[0])
        seqs.append(_elu(y) + pe_ref[t])
    x3 = seqs[T - 1]
    q3 = jnp.dot(x3, wqT_ref[...], preferred_element_type=_f32) + bq_ref[0]
    aw = []
    vs = []
    for t in range(T):
        kt = jnp.dot(seqs[t], wkT_ref[...],
                     preferred_element_type=_f32) + bk_ref[0]
        vs.append(jnp.dot(seqs[t], wvT_ref[...],
                          preferred_element_type=_f32) + bv_ref[0])
        aw.append(jnp.dot(q3 * kt, eh_ref[...],
                          preferred_element_type=_f32))
    mx = jnp.maximum(jnp.maximum(aw[0], aw[1]), jnp.maximum(aw[2], aw[3]))
    es = [jnp.exp(a - mx) for a in aw]
    den = es[0] + es[1] + es[2] + es[3]
    ao = None
    for t in range(T):
        w = es[t] / den
        wex = jnp.dot(w, eexp_ref[...], preferred_element_type=_f32)
        ao = wex * vs[t] if ao is None else ao + wex * vs[t]
    out = jnp.dot(ao, woutT_ref[...], preferred_element_type=_f32) + bout_ref[0]
    y2 = _layer_norm(x3 + out, ln2_s_ref[0], ln2_b_ref[0])
    z = _elu(y2)
    out_ref[...] = jnp.dot(z, wcT_ref[...],
                           preferred_element_type=_f32) + bc_ref[0]


def _run_tcc(msg, h0, pe, ln1_s2, ln1_b2, wqT, bq2, wkT, bk2, wvT, bv2,
             eh, eexp, woutT, bout2, ln2_s2, ln2_b2, wcT_pad, bc2_pad):
    full = lambda s: pl.BlockSpec(s, lambda nb: tuple(0 for _ in s))
    return pl.pallas_call(
        _tcc_body,
        grid=(NB,),
        in_specs=[
            pl.BlockSpec((2, T, BLK, HID), lambda nb: (0, 0, nb, 0)),
            pl.BlockSpec((T, BLK, HID), lambda nb: (0, nb, 0)),
            full((T, HID)),
            full((1, HID)), full((1, HID)),
            full((HID, HID)), full((1, HID)),
            full((HID, HID)), full((1, HID)),
            full((HID, HID)), full((1, HID)),
            full((HID, 16)), full((16, HID)),
            full((HID, HID)), full((1, HID)),
            full((1, HID)), full((1, HID)),
            full((HID, HID)), full((1, HID)),
        ],
        out_specs=pl.BlockSpec((BLK, HID), lambda nb: (nb, 0)),
        out_shape=jax.ShapeDtypeStruct((N, HID), _f32),
    )(msg, h0, pe, ln1_s2, ln1_b2, wqT, bq2, wkT, bk2, wvT, bv2,
      eh, eexp, woutT, bout2, ln2_s2, ln2_b2, wcT_pad, bc2_pad)


def _pos_enc():
    pos = jnp.arange(T, dtype=_f32)[:, None]
    div = jnp.exp(jnp.arange(0, HID, 2, dtype=_f32)
                  * (-math.log(10000.0) / HID))
    pe = jnp.zeros((T, HID), dtype=_f32)
    pe = pe.at[:, 0::2].set(jnp.sin(pos * div))
    pe = pe.at[:, 1::2].set(jnp.cos(pos * div))
    return pe


def kernel(x, edge_index, W_in, b_in, W_gat, a_gat, ln1_s, ln1_b, Wqkv, bqkv,
           Wout, bout, ln2_s, ln2_b, Wc, bc):
    winT = W_in.T
    wgatT = W_gat.T
    a1 = a_gat[0, :DH]
    a2 = a_gat[0, DH:]
    sel1 = jnp.kron(jnp.eye(H, dtype=_f32), a1[:, None])
    sel2 = jnp.kron(jnp.eye(H, dtype=_f32), a2[:, None])
    wsrc = wgatT @ jnp.concatenate([sel1, sel1], axis=1)
    wdst = wgatT @ jnp.concatenate([sel2, sel2], axis=1)

    ei = edge_index.astype(_i32)
    ei = jnp.pad(ei, ((0, 0), (0, 0), (0, EPAD - E)))
    ei_flat = ei.reshape(T * 2 * EPAD)

    h0, hh, ss, sd = _run_tca(x, winT, b_in[None, :], wgatT, wsrc, wdst)
    ssrc16 = ss.reshape(T * N, 16)
    sdst16 = sd.reshape(T * N, 16)
    hh_flat = hh.reshape(T * N, HID)

    cm = jnp.max(ss, axis=1) + jnp.max(sd, axis=1)
    cmax = jnp.where(cm > 0, cm, 0.2 * cm)

    den, ex = _run_sc1(ei_flat, ssrc16, sdst16, cmax)
    dsum = _run_densum(den.reshape(2, TNPAD, 16))

    msg = _run_sc2(ei_flat, ex, hh_flat, dsum)
    msg = msg.reshape(2, T, NPAD, HID)[:, :, :N]

    qs, ks_, vs_ = [], [], []
    for h in range(H):
        qs.append(Wqkv[h * 3 * DH: h * 3 * DH + DH])
        ks_.append(Wqkv[h * 3 * DH + DH: h * 3 * DH + 2 * DH])
        vs_.append(Wqkv[h * 3 * DH + 2 * DH: h * 3 * DH + 3 * DH])
    wq = jnp.concatenate(qs, axis=0)
    wk = jnp.concatenate(ks_, axis=0)
    wv = jnp.concatenate(vs_, axis=0)
    bqkv3 = bqkv.reshape(H, 3 * DH)
    bq = bqkv3[:, :DH].reshape(HID)
    bk = bqkv3[:, DH:2 * DH].reshape(HID)
    bv = bqkv3[:, 2 * DH:].reshape(HID)

    eh = jnp.kron(jnp.eye(H, dtype=_f32), jnp.ones((DH, 1), _f32))
    eh16 = jnp.concatenate([eh, eh], axis=1) / math.sqrt(DH)
    eexp = jnp.concatenate([eh, eh], axis=1).T * 0.5

    wcT_pad = jnp.zeros((HID, HID), _f32).at[:, :NCLS].set(Wc.T)
    bc_pad = jnp.zeros((HID,), _f32).at[:NCLS].set(bc)

    logits = _run_tcc(msg, h0, _pos_enc(), ln1_s[None, :], ln1_b[None, :],
                      wq.T, bq[None, :], wk.T, bk[None, :], wv.T, bv[None, :],
                      eh16, eexp, Wout.T, bout[None, :],
                      ln2_s[None, :], ln2_b[None, :], wcT_pad, bc_pad[None, :])
    return logits[:, :NCLS]


# speedup vs baseline: 54.5342x; 1.0901x over previous
import functools
import math

import jax
import jax.numpy as jnp
from jax import lax
from jax.experimental import pallas as pl
from jax.experimental.pallas import tpu as pltpu
from jax.experimental.pallas import tpu_sc as plsc

N = 10000
E = 160000
T = 4
IN_DIM = 128
HID = 128
H = 8
DH = 16
NCLS = 40

NWORK = 32
EPAD = 163840
EW = EPAD // NWORK
CH1 = 320
CH2 = 64
NBUF2 = 4
BLK = 1000
NB = N // BLK
RPT1 = 2504
TNPAD = 16 * RPT1
RPT2 = 632
NPAD = 16 * RPT2

_f32 = jnp.float32
_i32 = jnp.int32
_GATHER_DNUMS = lax.GatherDimensionNumbers(
    offset_dims=(), collapsed_slice_dims=(0,), start_index_map=(0,))


def _tca_body(x_ref, winT_ref, b_in_ref, wgatT_ref, wsrc_ref, wdst_ref,
              h0_ref, hh_ref, ss_ref, sd_ref):
    xb = x_ref[0]
    h0 = jnp.dot(xb, winT_ref[...], preferred_element_type=_f32) + b_in_ref[0]
    hh = jnp.dot(h0, wgatT_ref[...], preferred_element_type=_f32)
    ss = jnp.dot(h0, wsrc_ref[...], preferred_element_type=_f32)
    sd = jnp.dot(h0, wdst_ref[...], preferred_element_type=_f32)
    h0_ref[0] = h0
    hh_ref[0] = hh
    ss_ref[0] = ss
    sd_ref[0] = sd


def _run_tca(x, winT, b_in2, wgatT, wsrc, wdst):
    full = lambda s: pl.BlockSpec(s, lambda t, nb: tuple(0 for _ in s))
    return pl.pallas_call(
        _tca_body,
        grid=(T, NB),
        in_specs=[
            pl.BlockSpec((1, BLK, IN_DIM), lambda t, nb: (t, nb, 0)),
            full((IN_DIM, HID)),
            full((1, HID)),
            full((HID, HID)),
            full((HID, 16)),
            full((HID, 16)),
        ],
        out_specs=[
            pl.BlockSpec((1, BLK, HID), lambda t, nb: (t, nb, 0)),
            pl.BlockSpec((1, BLK, HID), lambda t, nb: (t, nb, 0)),
            pl.BlockSpec((1, BLK, 16), lambda t, nb: (t, nb, 0)),
            pl.BlockSpec((1, BLK, 16), lambda t, nb: (t, nb, 0)),
        ],
        out_shape=[
            jax.ShapeDtypeStruct((T, N, HID), _f32),
            jax.ShapeDtypeStruct((T, N, HID), _f32),
            jax.ShapeDtypeStruct((T, N, 16), _f32),
            jax.ShapeDtypeStruct((T, N, 16), _f32),
        ],
    )(x, winT, b_in2, wgatT, wsrc, wdst)


def _sc1_body(ei_ref, ssrc_ref, sdst_ref, c_ref, den_ref, ex_ref,
              srco, dsto, srow, drow, exb, cbuf, denom_sp,
              gs0, gs1, gs2, gs3, ss0, ss1, ss2, ss3):
    cid = lax.axis_index("c")
    sid = lax.axis_index("s")
    wid = sid * 2 + cid
    pltpu.sync_copy(c_ref, cbuf)
    gsems = [gs0, gs1, gs2, gs3]
    ssems = [ss0, ss1, ss2, ss3]
    NCH = EW // CH1
    NP = NCH // NBUF2
    zv = jnp.zeros((16,), _f32)

    def zbuf(g, _):
        exb[0, g] = zv
        return 0

    lax.fori_loop(0, CH1, zbuf, 0)
    for z in range(RPT1 // CH1):
        pltpu.sync_copy(exb.at[0],
                        denom_sp.at[pl.ds(sid * RPT1 + z * CH1, CH1)])
    zrem = RPT1 - (RPT1 // CH1) * CH1
    if zrem:
        pltpu.sync_copy(
            exb.at[0].at[pl.ds(0, zrem)],
            denom_sp.at[pl.ds(sid * RPT1 + (RPT1 // CH1) * CH1, zrem)])
    plsc.subcore_barrier()

    def load_fire(t, j, u):
        base = wid * EW + j * CH1
        pltpu.sync_copy(ei_ref.at[pl.ds((t * 2) * EPAD + base, CH1)],
                        srco.at[u])
        pltpu.sync_copy(ei_ref.at[pl.ds((t * 2 + 1) * EPAD + base, CH1)],
                        dsto.at[u])

        def addoff(g, _):
            srco[u, pl.ds(g * 16, 16)] = srco[u, pl.ds(g * 16, 16)] + t * N
            dsto[u, pl.ds(g * 16, 16)] = dsto[u, pl.ds(g * 16, 16)] + t * N
            return 0

        lax.fori_loop(0, CH1 // 16, addoff, 0)
        pltpu.async_copy(ssrc_ref.at[srco.at[u]], srow.at[u], gsems[u])
        pltpu.async_copy(sdst_ref.at[dsto.at[u]], drow.at[u], gsems[u])

    def wait_gathers(u):
        pltpu.make_async_copy(ssrc_ref.at[srco.at[u]], srow.at[u],
                              gsems[u]).wait()
        pltpu.make_async_copy(sdst_ref.at[dsto.at[u]], drow.at[u],
                              gsems[u]).wait()

    def wait_stores(u):
        pltpu.make_async_copy(exb.at[u], denom_sp.at[pl.ds(0, CH1)],
                              ssems[u]).wait()

    def compute(t, j, u, cvec):
        base = wid * EW + j * CH1

        def body(e, _):
            s = srow[u, e] + drow[u, e]
            s = jnp.where(s > 0, s, 0.2 * s) - cvec
            ex = jnp.exp(s)
            valid = jnp.where(base + e < E, 1.0, 0.0).astype(_f32)
            exb[u, e] = ex * valid
            return 0

        lax.fori_loop(0, CH1, body, 0)

    for t in range(T):
        cvec = cbuf[t]
        load_fire(t, 0, 0)

        def quad(g, _, t=t, cvec=cvec):
            for u in range(NBUF2):
                j = g * NBUF2 + u
                nu = (u + 1) % NBUF2
                if u < NBUF2 - 1:
                    @pl.when(g > 0)
                    def _(nu=nu):
                        wait_stores(nu)

                    load_fire(t, j + 1, nu)
                else:
                    @pl.when(g < NP - 1)
                    def _(j=j):
                        wait_stores(0)
                        load_fire(t, j + 1, 0)
                wait_gathers(u)
                compute(t, j, u, cvec)
                base = wid * EW + j * CH1
                pltpu.sync_copy(exb.at[u],
                                ex_ref.at[pl.ds(t * EPAD + base, CH1)])
                pltpu.async_copy(exb.at[u], denom_sp.at[dsto.at[u]],
                                 ssems[u], add=True)
            return 0

        lax.fori_loop(0, NP, quad, 0)
        for u in range(NBUF2):
            wait_stores(u)

    plsc.subcore_barrier()
    pltpu.sync_copy(
        denom_sp.at[pl.ds(sid * RPT1, RPT1)],
        den_ref.at[pl.ds(cid * TNPAD + sid * RPT1, RPT1)])


def _run_sc1(ei_flat, ssrc16, sdst16, cmax):
    mesh = plsc.VectorSubcoreMesh(core_axis_name="c", subcore_axis_name="s")
    k = pl.kernel(
        _sc1_body,
        out_type=[jax.ShapeDtypeStruct((2 * TNPAD, 16), _f32),
                  jax.ShapeDtypeStruct((T * EPAD, 16), _f32)],
        mesh=mesh,
        scratch_types=[
            pltpu.VMEM((NBUF2, CH1), _i32),
            pltpu.VMEM((NBUF2, CH1), _i32),
            pltpu.VMEM((NBUF2, CH1, 16), _f32),
            pltpu.VMEM((NBUF2, CH1, 16), _f32),
            pltpu.VMEM((NBUF2, CH1, 16), _f32),
            pltpu.VMEM((T, 16), _f32),
            pltpu.VMEM_SHARED((TNPAD, 16), _f32),
            pltpu.SemaphoreType.DMA,
            pltpu.SemaphoreType.DMA,
            pltpu.SemaphoreType.DMA,
            pltpu.SemaphoreType.DMA,
            pltpu.SemaphoreType.DMA,
            pltpu.SemaphoreType.DMA,
            pltpu.SemaphoreType.DMA,
            pltpu.SemaphoreType.DMA,
        ],
        compiler_params=pltpu.CompilerParams(use_tc_tiling_on_sc=False),
    )
    return k(ei_flat, ssrc16, sdst16, cmax)


def _densum_body(den_ref, out_ref):
    out_ref[...] = 1.0 / (den_ref[0] + den_ref[1] + 1e-16)


def _run_densum(den2):
    return pl.pallas_call(
        _densum_body,
        grid=(16,),
        in_specs=[pl.BlockSpec((2, RPT1, 16), lambda nb: (0, nb, 0))],
        out_specs=pl.BlockSpec((RPT1, 16), lambda nb: (nb, 0)),
        out_shape=jax.ShapeDtypeStruct((TNPAD, 16), _f32),
    )(den2)


def _sc2_body(ei_ref, ex_ref, hh_ref, dA_ref, msg_ref,
              srco, dstl, dsto, hhb, exb, dab, out_sp,
              gs0, gs1, gs2, gs3, ss0, ss1, ss2, ss3):
    cid = lax.axis_index("c")
    sid = lax.axis_index("s")
    wid = sid * 2 + cid
    zv = jnp.zeros((16,), _f32)
    splats = [jnp.full((16,), h, _i32) for h in range(H)]
    gsems = [gs0, gs1, gs2, gs3]
    ssems = [ss0, ss1, ss2, ss3]
    NCH = EW // CH2
    NP = NCH // NBUF2

    def load_fire(t, j, u):
        base = wid * EW + j * CH2
        pltpu.sync_copy(ei_ref.at[pl.ds((t * 2) * EPAD + base, CH2)],
                        srco.at[u])
        pltpu.sync_copy(ei_ref.at[pl.ds((t * 2 + 1) * EPAD + base, CH2)],
                        dstl.at[u])

        def addoff(g, _):
            srco[u, pl.ds(g * 16, 16)] = srco[u, pl.ds(g * 16, 16)] + t * N
            dsto[u, pl.ds(g * 16, 16)] = dstl[u, pl.ds(g * 16, 16)] + t * N
            return 0

        lax.fori_loop(0, CH2 // 16, addoff, 0)
        pltpu.async_copy(hh_ref.at[srco.at[u]], hhb.at[u], gsems[u])
        pltpu.async_copy(ex_ref.at[pl.ds(t * EPAD + base, CH2)],
                         exb.at[u], gsems[u])
        pltpu.async_copy(dA_ref.at[dsto.at[u]], dab.at[u], gsems[u])

    def wait_gathers(t, j, u):
        base = wid * EW + j * CH2
        pltpu.make_async_copy(hh_ref.at[srco.at[u]], hhb.at[u],
                              gsems[u]).wait()
        pltpu.make_async_copy(ex_ref.at[pl.ds(t * EPAD + base, CH2)],
                              exb.at[u], gsems[u]).wait()
        pltpu.make_async_copy(dA_ref.at[dsto.at[u]], dab.at[u],
                              gsems[u]).wait()

    def wait_scatter(u):
        pltpu.make_async_copy(hhb.at[u], out_sp.at[pl.ds(0, CH2)],
                              ssems[u]).wait()

    def compute(u):
        def body(e, _):
            al = exb[u, e] * dab[u, e]
            for h in range(H):
                av = lax.gather(
                    al, splats[h][:, None], _GATHER_DNUMS, (1,),
                    mode=lax.GatherScatterMode.PROMISE_IN_BOUNDS)
                hv = hhb[u, e, pl.ds(h * 16, 16)]
                hhb[u, e, pl.ds(h * 16, 16)] = hv * av
            return 0

        lax.fori_loop(0, CH2, body, 0)

    for t in range(T):
        def zbuf(g, _):
            for kk in range(H):
                hhb[0, g, pl.ds(kk * 16, 16)] = zv
            return 0

        lax.fori_loop(0, CH2, zbuf, 0)
        for z in range(RPT2 // CH2):
            pltpu.sync_copy(hhb.at[0],
                            out_sp.at[pl.ds(sid * RPT2 + z * CH2, CH2)])
        zrem = RPT2 - (RPT2 // CH2) * CH2
        if zrem:
            pltpu.sync_copy(
                hhb.at[0].at[pl.ds(0, zrem)],
                out_sp.at[pl.ds(sid * RPT2 + (RPT2 // CH2) * CH2, zrem)])
        plsc.subcore_barrier()

        load_fire(t, 0, 0)

        def quad(g, _, t=t):
            for u in range(NBUF2):
                j = g * NBUF2 + u
                nu = (u + 1) % NBUF2
                if u < NBUF2 - 1:
                    @pl.when(g > 0)
                    def _(nu=nu):
                        wait_scatter(nu)

                    load_fire(t, j + 1, nu)
                else:
                    @pl.when(g < NP - 1)
                    def _(j=j):
                        wait_scatter(0)
                        load_fire(t, j + 1, 0)
                wait_gathers(t, j, u)
                compute(u)
                pltpu.async_copy(hhb.at[u], out_sp.at[dstl.at[u]],
                                 ssems[u], add=True)
            return 0

        lax.fori_loop(0, NP, quad, 0)
        for u in range(NBUF2):
            wait_scatter(u)
        plsc.subcore_barrier()
        pltpu.sync_copy(
            out_sp.at[pl.ds(sid * RPT2, RPT2)],
            msg_ref.at[pl.ds(cid * (T * NPAD) + t * NPAD + sid * RPT2,
                             RPT2)])
        plsc.subcore_barrier()


def _run_sc2(ei_flat, ex, hh_flat, dA):
    mesh = plsc.VectorSubcoreMesh(core_axis_name="c", subcore_axis_name="s")
    k = pl.kernel(
        _sc2_body,
        out_type=jax.ShapeDtypeStruct((2 * T * NPAD, HID), _f32),
        mesh=mesh,
        scratch_types=[
            pltpu.VMEM((NBUF2, CH2), _i32),
            pltpu.VMEM((NBUF2, CH2), _i32),
            pltpu.VMEM((NBUF2, CH2), _i32),
            pltpu.VMEM((NBUF2, CH2, HID), _f32),
            pltpu.VMEM((NBUF2, CH2, 16), _f32),
            pltpu.VMEM((NBUF2, CH2, 16), _f32),
            pltpu.VMEM_SHARED((NPAD, HID), _f32),
            pltpu.SemaphoreType.DMA,
            pltpu.SemaphoreType.DMA,
            pltpu.SemaphoreType.DMA,
            pltpu.SemaphoreType.DMA,
            pltpu.SemaphoreType.DMA,
            pltpu.SemaphoreType.DMA,
            pltpu.SemaphoreType.DMA,
            pltpu.SemaphoreType.DMA,
        ],
        compiler_params=pltpu.CompilerParams(use_tc_tiling_on_sc=False),
    )
    return k(ei_flat, ex, hh_flat, dA)


def _layer_norm(m, s, b):
    mu = jnp.mean(m, axis=-1, keepdims=True)
    var = jnp.mean((m - mu) ** 2, axis=-1, keepdims=True)
    return (m - mu) * lax.rsqrt(var + 1e-5) * s + b


def _elu(y):
    return jnp.where(y > 0, y, jnp.exp(y) - 1.0)


def _tcc_body(msg_ref, h0_ref, pe_ref, ln1_s_ref, ln1_b_ref,
              wqT_ref, bq_ref, wkT_ref, bk_ref, wvT_ref, bv_ref,
              eh_ref, eexp_ref, woutT_ref, bout_ref,
              ln2_s_ref, ln2_b_ref, wcT_ref, bc_ref, out_ref):
    seqs = []
    for t in range(T):
        m = msg_ref[0, t] + msg_ref[1, t] + h0_ref[t]
        y = _layer_norm(m, ln1_s_ref[0], ln1_b_ref[0])
        seqs.append(_elu(y) + pe_ref[t])
    x3 = seqs[T - 1]
    q3 = jnp.dot(x3, wqT_ref[...], preferred_element_type=_f32) + bq_ref[0]
    aw = []
    vs = []
    for t in range(T):
        kt = jnp.dot(seqs[t], wkT_ref[...],
                     preferred_element_type=_f32) + bk_ref[0]
        vs.append(jnp.dot(seqs[t], wvT_ref[...],
                          preferred_element_type=_f32) + bv_ref[0])
        aw.append(jnp.dot(q3 * kt, eh_ref[...],
                          preferred_element_type=_f32))
    mx = jnp.maximum(jnp.maximum(aw[0], aw[1]), jnp.maximum(aw[2], aw[3]))
    es = [jnp.exp(a - mx) for a in aw]
    den = es[0] + es[1] + es[2] + es[3]
    ao = None
    for t in range(T):
        w = es[t] / den
        wex = jnp.dot(w, eexp_ref[...], preferred_element_type=_f32)
        ao = wex * vs[t] if ao is None else ao + wex * vs[t]
    out = jnp.dot(ao, woutT_ref[...], preferred_element_type=_f32) + bout_ref[0]
    y2 = _layer_norm(x3 + out, ln2_s_ref[0], ln2_b_ref[0])
    z = _elu(y2)
    out_ref[...] = jnp.dot(z, wcT_ref[...],
                           preferred_element_type=_f32) + bc_ref[0]


def _run_tcc(msg, h0, pe, ln1_s2, ln1_b2, wqT, bq2, wkT, bk2, wvT, bv2,
             eh, eexp, woutT, bout2, ln2_s2, ln2_b2, wcT_pad, bc2_pad):
    full = lambda s: pl.BlockSpec(s, lambda nb: tuple(0 for _ in s))
    return pl.pallas_call(
        _tcc_body,
        grid=(NB,),
        in_specs=[
            pl.BlockSpec((2, T, BLK, HID), lambda nb: (0, 0, nb, 0)),
            pl.BlockSpec((T, BLK, HID), lambda nb: (0, nb, 0)),
            full((T, HID)),
            full((1, HID)), full((1, HID)),
            full((HID, HID)), full((1, HID)),
            full((HID, HID)), full((1, HID)),
            full((HID, HID)), full((1, HID)),
            full((HID, 16)), full((16, HID)),
            full((HID, HID)), full((1, HID)),
            full((1, HID)), full((1, HID)),
            full((HID, HID)), full((1, HID)),
        ],
        out_specs=pl.BlockSpec((BLK, HID), lambda nb: (nb, 0)),
        out_shape=jax.ShapeDtypeStruct((N, HID), _f32),
    )(msg, h0, pe, ln1_s2, ln1_b2, wqT, bq2, wkT, bk2, wvT, bv2,
      eh, eexp, woutT, bout2, ln2_s2, ln2_b2, wcT_pad, bc2_pad)


def _pos_enc():
    pos = jnp.arange(T, dtype=_f32)[:, None]
    div = jnp.exp(jnp.arange(0, HID, 2, dtype=_f32)
                  * (-math.log(10000.0) / HID))
    pe = jnp.zeros((T, HID), dtype=_f32)
    pe = pe.at[:, 0::2].set(jnp.sin(pos * div))
    pe = pe.at[:, 1::2].set(jnp.cos(pos * div))
    return pe


def kernel(x, edge_index, W_in, b_in, W_gat, a_gat, ln1_s, ln1_b, Wqkv, bqkv,
           Wout, bout, ln2_s, ln2_b, Wc, bc):
    winT = W_in.T
    wgatT = W_gat.T
    a1 = a_gat[0, :DH]
    a2 = a_gat[0, DH:]
    sel1 = jnp.kron(jnp.eye(H, dtype=_f32), a1[:, None])
    sel2 = jnp.kron(jnp.eye(H, dtype=_f32), a2[:, None])
    wsrc = wgatT @ jnp.concatenate([sel1, sel1], axis=1)
    wdst = wgatT @ jnp.concatenate([sel2, sel2], axis=1)

    ei = edge_index.astype(_i32)
    ei = jnp.pad(ei, ((0, 0), (0, 0), (0, EPAD - E)))
    ei_flat = ei.reshape(T * 2 * EPAD)

    h0, hh, ss, sd = _run_tca(x, winT, b_in[None, :], wgatT, wsrc, wdst)
    ssrc16 = ss.reshape(T * N, 16)
    sdst16 = sd.reshape(T * N, 16)
    hh_flat = hh.reshape(T * N, HID)

    cm = jnp.max(ss, axis=1) + jnp.max(sd, axis=1)
    cmax = jnp.where(cm > 0, cm, 0.2 * cm)

    den, ex = _run_sc1(ei_flat, ssrc16, sdst16, cmax)
    dsum = _run_densum(den.reshape(2, TNPAD, 16))

    msg = _run_sc2(ei_flat, ex, hh_flat, dsum)
    msg = msg.reshape(2, T, NPAD, HID)[:, :, :N]

    qs, ks_, vs_ = [], [], []
    for h in range(H):
        qs.append(Wqkv[h * 3 * DH: h * 3 * DH + DH])
        ks_.append(Wqkv[h * 3 * DH + DH: h * 3 * DH + 2 * DH])
        vs_.append(Wqkv[h * 3 * DH + 2 * DH: h * 3 * DH + 3 * DH])
    wq = jnp.concatenate(qs, axis=0)
    wk = jnp.concatenate(ks_, axis=0)
    wv = jnp.concatenate(vs_, axis=0)
    bqkv3 = bqkv.reshape(H, 3 * DH)
    bq = bqkv3[:, :DH].reshape(HID)
    bk = bqkv3[:, DH:2 * DH].reshape(HID)
    bv = bqkv3[:, 2 * DH:].reshape(HID)

    eh = jnp.kron(jnp.eye(H, dtype=_f32), jnp.ones((DH, 1), _f32))
    eh16 = jnp.concatenate([eh, eh], axis=1) / math.sqrt(DH)
    eexp = jnp.concatenate([eh, eh], axis=1).T * 0.5

    wcT_pad = jnp.zeros((HID, HID), _f32).at[:, :NCLS].set(Wc.T)
    bc_pad = jnp.zeros((HID,), _f32).at[:NCLS].set(bc)

    logits = _run_tcc(msg, h0, _pos_enc(), ln1_s[None, :], ln1_b[None, :],
                      wq.T, bq[None, :], wk.T, bk[None, :], wv.T, bv[None, :],
                      eh16, eexp, Wout.T, bout[None, :],
                      ln2_s[None, :], ln2_b[None, :], wcT_pad, bc_pad[None, :])
    return logits[:, :NCLS]


# speedup vs baseline: 55.4824x; 1.0174x over previous
import functools
import math

import jax
import jax.numpy as jnp
from jax import lax
from jax.experimental import pallas as pl
from jax.experimental.pallas import tpu as pltpu
from jax.experimental.pallas import tpu_sc as plsc

N = 10000
E = 160000
T = 4
IN_DIM = 128
HID = 128
H = 8
DH = 16
NCLS = 40

NWORK = 32
EPAD = 163840
EW = EPAD // NWORK
CH1 = 320
CH2 = 64
NBUF2 = 4
BLK = 1000
NB = N // BLK
RPT1 = 2504
TNPAD = 16 * RPT1
RPT2 = 632
NPAD = 16 * RPT2

_f32 = jnp.float32
_i32 = jnp.int32
_GATHER_DNUMS = lax.GatherDimensionNumbers(
    offset_dims=(), collapsed_slice_dims=(0,), start_index_map=(0,))


def _tca_body(x_ref, winT_ref, b_in_ref, wgatT_ref, wsrc_ref, wdst_ref,
              h0_ref, hh_ref, ss_ref, sd_ref):
    xb = x_ref[0]
    h0 = jnp.dot(xb, winT_ref[...], preferred_element_type=_f32) + b_in_ref[0]
    hh = jnp.dot(h0, wgatT_ref[...], preferred_element_type=_f32)
    ss = jnp.dot(h0, wsrc_ref[...], preferred_element_type=_f32)
    sd = jnp.dot(h0, wdst_ref[...], preferred_element_type=_f32)
    h0_ref[0] = h0
    hh_ref[0] = hh
    ss_ref[0] = ss
    sd_ref[0] = sd


def _run_tca(x, winT, b_in2, wgatT, wsrc, wdst):
    full = lambda s: pl.BlockSpec(s, lambda t, nb: tuple(0 for _ in s))
    return pl.pallas_call(
        _tca_body,
        grid=(T, NB),
        in_specs=[
            pl.BlockSpec((1, BLK, IN_DIM), lambda t, nb: (t, nb, 0)),
            full((IN_DIM, HID)),
            full((1, HID)),
            full((HID, HID)),
            full((HID, 16)),
            full((HID, 16)),
        ],
        out_specs=[
            pl.BlockSpec((1, BLK, HID), lambda t, nb: (t, nb, 0)),
            pl.BlockSpec((1, BLK, HID), lambda t, nb: (t, nb, 0)),
            pl.BlockSpec((1, BLK, 16), lambda t, nb: (t, nb, 0)),
            pl.BlockSpec((1, BLK, 16), lambda t, nb: (t, nb, 0)),
        ],
        out_shape=[
            jax.ShapeDtypeStruct((T, N, HID), _f32),
            jax.ShapeDtypeStruct((T, N, HID), _f32),
            jax.ShapeDtypeStruct((T, N, 16), _f32),
            jax.ShapeDtypeStruct((T, N, 16), _f32),
        ],
    )(x, winT, b_in2, wgatT, wsrc, wdst)


def _sc1_body(ei_ref, ssrc_ref, sdst_ref, c_ref, den_ref, ex_ref,
              srco, dsto, srow, drow, exb, cbuf, denom_sp,
              gs0, gs1, gs2, gs3, ss0, ss1, ss2, ss3):
    cid = lax.axis_index("c")
    sid = lax.axis_index("s")
    wid = sid * 2 + cid
    pltpu.sync_copy(c_ref, cbuf)
    gsems = [gs0, gs1, gs2, gs3]
    ssems = [ss0, ss1, ss2, ss3]
    NCH = EW // CH1
    NP = NCH // NBUF2
    zv = jnp.zeros((16,), _f32)

    def zbuf(g, _):
        exb[0, g] = zv
        return 0

    lax.fori_loop(0, CH1, zbuf, 0)
    for z in range(RPT1 // CH1):
        pltpu.sync_copy(exb.at[0],
                        denom_sp.at[pl.ds(sid * RPT1 + z * CH1, CH1)])
    zrem = RPT1 - (RPT1 // CH1) * CH1
    if zrem:
        pltpu.sync_copy(
            exb.at[0].at[pl.ds(0, zrem)],
            denom_sp.at[pl.ds(sid * RPT1 + (RPT1 // CH1) * CH1, zrem)])
    plsc.subcore_barrier()

    def load_fire(t, j, u):
        base = wid * EW + j * CH1
        pltpu.sync_copy(ei_ref.at[pl.ds((t * 2) * EPAD + base, CH1)],
                        srco.at[u])
        pltpu.sync_copy(ei_ref.at[pl.ds((t * 2 + 1) * EPAD + base, CH1)],
                        dsto.at[u])

        def addoff(g, _):
            srco[u, pl.ds(g * 16, 16)] = srco[u, pl.ds(g * 16, 16)] + t * N
            dsto[u, pl.ds(g * 16, 16)] = dsto[u, pl.ds(g * 16, 16)] + t * N
            return 0

        lax.fori_loop(0, CH1 // 16, addoff, 0)
        pltpu.async_copy(ssrc_ref.at[srco.at[u]], srow.at[u], gsems[u])
        pltpu.async_copy(sdst_ref.at[dsto.at[u]], drow.at[u], gsems[u])

    def wait_gathers(u):
        pltpu.make_async_copy(ssrc_ref.at[srco.at[u]], srow.at[u],
                              gsems[u]).wait()
        pltpu.make_async_copy(sdst_ref.at[dsto.at[u]], drow.at[u],
                              gsems[u]).wait()

    def wait_stores(u):
        pltpu.make_async_copy(exb.at[u], denom_sp.at[pl.ds(0, CH1)],
                              ssems[u]).wait()

    def compute(t, j, u, cvec):
        base = wid * EW + j * CH1

        def body(e, _):
            s = srow[u, e] + drow[u, e]
            s = jnp.where(s > 0, s, 0.2 * s) - cvec
            ex = jnp.exp(s)
            valid = jnp.where(base + e < E, 1.0, 0.0).astype(_f32)
            exb[u, e] = ex * valid
            return 0

        lax.fori_loop(0, CH1, body, 0)

    for t in range(T):
        cvec = cbuf[t]
        load_fire(t, 0, 0)

        def quad(g, _, t=t, cvec=cvec):
            for u in range(NBUF2):
                j = g * NBUF2 + u
                nu = (u + 1) % NBUF2
                if u < NBUF2 - 1:
                    @pl.when(g > 0)
                    def _(nu=nu):
                        wait_stores(nu)

                    load_fire(t, j + 1, nu)
                else:
                    @pl.when(g < NP - 1)
                    def _(j=j):
                        wait_stores(0)
                        load_fire(t, j + 1, 0)
                wait_gathers(u)
                compute(t, j, u, cvec)
                base = wid * EW + j * CH1
                pltpu.sync_copy(exb.at[u],
                                ex_ref.at[pl.ds(t * EPAD + base, CH1)])
                pltpu.async_copy(exb.at[u], denom_sp.at[dsto.at[u]],
                                 ssems[u], add=True)
            return 0

        lax.fori_loop(0, NP, quad, 0)
        for u in range(NBUF2):
            wait_stores(u)

    plsc.subcore_barrier()
    pltpu.sync_copy(
        denom_sp.at[pl.ds(sid * RPT1, RPT1)],
        den_ref.at[pl.ds(cid * TNPAD + sid * RPT1, RPT1)])


def _run_sc1(ei_flat, ssrc16, sdst16, cmax):
    mesh = plsc.VectorSubcoreMesh(core_axis_name="c", subcore_axis_name="s")
    k = pl.kernel(
        _sc1_body,
        out_type=[jax.ShapeDtypeStruct((2 * TNPAD, 16), _f32),
                  jax.ShapeDtypeStruct((T * EPAD, 16), _f32)],
        mesh=mesh,
        scratch_types=[
            pltpu.VMEM((NBUF2, CH1), _i32),
            pltpu.VMEM((NBUF2, CH1), _i32),
            pltpu.VMEM((NBUF2, CH1, 16), _f32),
            pltpu.VMEM((NBUF2, CH1, 16), _f32),
            pltpu.VMEM((NBUF2, CH1, 16), _f32),
            pltpu.VMEM((T, 16), _f32),
            pltpu.VMEM_SHARED((TNPAD, 16), _f32),
            pltpu.SemaphoreType.DMA,
            pltpu.SemaphoreType.DMA,
            pltpu.SemaphoreType.DMA,
            pltpu.SemaphoreType.DMA,
            pltpu.SemaphoreType.DMA,
            pltpu.SemaphoreType.DMA,
            pltpu.SemaphoreType.DMA,
            pltpu.SemaphoreType.DMA,
        ],
        compiler_params=pltpu.CompilerParams(use_tc_tiling_on_sc=False),
    )
    return k(ei_flat, ssrc16, sdst16, cmax)


def _densum_body(den_ref, out_ref):
    out_ref[...] = 1.0 / (den_ref[0] + den_ref[1] + 1e-16)


def _run_densum(den2):
    return pl.pallas_call(
        _densum_body,
        grid=(16,),
        in_specs=[pl.BlockSpec((2, RPT1, 16), lambda nb: (0, nb, 0))],
        out_specs=pl.BlockSpec((RPT1, 16), lambda nb: (nb, 0)),
        out_shape=jax.ShapeDtypeStruct((TNPAD, 16), _f32),
    )(den2)


def _sc2_body(ei_ref, ex_ref, hh_ref, dA_ref, msg_ref,
              srco, dstl, dsto, hhb, exb, dab, out_sp,
              gs0, gs1, gs2, gs3, ss0, ss1, ss2, ss3):
    cid = lax.axis_index("c")
    sid = lax.axis_index("s")
    wid = sid * 2 + cid
    zv = jnp.zeros((16,), _f32)
    splats = [jnp.full((16,), h, _i32) for h in range(H)]
    gsems = [gs0, gs1, gs2, gs3]
    ssems = [ss0, ss1, ss2, ss3]
    NCH = EW // CH2
    NP = NCH // NBUF2

    def load_fire(t, j, u):
        base = wid * EW + j * CH2
        pltpu.sync_copy(ei_ref.at[pl.ds((t * 2) * EPAD + base, CH2)],
                        srco.at[u])
        pltpu.sync_copy(ei_ref.at[pl.ds((t * 2 + 1) * EPAD + base, CH2)],
                        dstl.at[u])

        def addoff(g, _):
            srco[u, pl.ds(g * 16, 16)] = srco[u, pl.ds(g * 16, 16)] + t * N
            dsto[u, pl.ds(g * 16, 16)] = dstl[u, pl.ds(g * 16, 16)] + t * N
            return 0

        lax.fori_loop(0, CH2 // 16, addoff, 0)
        pltpu.async_copy(hh_ref.at[srco.at[u]], hhb.at[u], gsems[u])
        pltpu.async_copy(ex_ref.at[pl.ds(t * EPAD + base, CH2)],
                         exb.at[u], gsems[u])
        pltpu.async_copy(dA_ref.at[dsto.at[u]], dab.at[u], gsems[u])

    def wait_gathers(t, j, u):
        base = wid * EW + j * CH2
        pltpu.make_async_copy(hh_ref.at[srco.at[u]], hhb.at[u],
                              gsems[u]).wait()
        pltpu.make_async_copy(ex_ref.at[pl.ds(t * EPAD + base, CH2)],
                              exb.at[u], gsems[u]).wait()
        pltpu.make_async_copy(dA_ref.at[dsto.at[u]], dab.at[u],
                              gsems[u]).wait()

    def wait_scatter(u):
        pltpu.make_async_copy(hhb.at[u], out_sp.at[pl.ds(0, CH2)],
                              ssems[u]).wait()

    def compute(u):
        def body(e, _):
            al = exb[u, e] * dab[u, e]
            for h in range(H):
                av = lax.gather(
                    al, splats[h][:, None], _GATHER_DNUMS, (1,),
                    mode=lax.GatherScatterMode.PROMISE_IN_BOUNDS)
                hv = hhb[u, e, pl.ds(h * 16, 16)]
                hhb[u, e, pl.ds(h * 16, 16)] = hv * av
            return 0

        lax.fori_loop(0, CH2, body, 0)

    for t in range(T):
        def zbuf(g, _):
            for kk in range(H):
                hhb[0, g, pl.ds(kk * 16, 16)] = zv
            return 0

        lax.fori_loop(0, CH2, zbuf, 0)
        for z in range(RPT2 // CH2):
            pltpu.sync_copy(hhb.at[0],
                            out_sp.at[pl.ds(sid * RPT2 + z * CH2, CH2)])
        zrem = RPT2 - (RPT2 // CH2) * CH2
        if zrem:
            pltpu.sync_copy(
                hhb.at[0].at[pl.ds(0, zrem)],
                out_sp.at[pl.ds(sid * RPT2 + (RPT2 // CH2) * CH2, zrem)])
        plsc.subcore_barrier()

        load_fire(t, 0, 0)

        def quad(g, _, t=t):
            for u in range(NBUF2):
                j = g * NBUF2 + u
                nu = (u + 1) % NBUF2
                if u < NBUF2 - 1:
                    @pl.when(g > 0)
                    def _(nu=nu):
                        wait_scatter(nu)

                    load_fire(t, j + 1, nu)
                else:
                    @pl.when(g < NP - 1)
                    def _(j=j):
                        wait_scatter(0)
                        load_fire(t, j + 1, 0)
                wait_gathers(t, j, u)
                compute(u)
                pltpu.async_copy(hhb.at[u], out_sp.at[dstl.at[u]],
                                 ssems[u], add=True)
            return 0

        lax.fori_loop(0, NP, quad, 0)
        for u in range(NBUF2):
            wait_scatter(u)
        plsc.subcore_barrier()
        pltpu.sync_copy(
            out_sp.at[pl.ds(sid * RPT2, RPT2)],
            msg_ref.at[pl.ds(cid * (T * NPAD) + t * NPAD + sid * RPT2,
                             RPT2)])
        plsc.subcore_barrier()


def _run_sc2(ei_flat, ex, hh_flat, dA):
    mesh = plsc.VectorSubcoreMesh(core_axis_name="c", subcore_axis_name="s")
    k = pl.kernel(
        _sc2_body,
        out_type=jax.ShapeDtypeStruct((2 * T * NPAD, HID), _f32),
        mesh=mesh,
        scratch_types=[
            pltpu.VMEM((NBUF2, CH2), _i32),
            pltpu.VMEM((NBUF2, CH2), _i32),
            pltpu.VMEM((NBUF2, CH2), _i32),
            pltpu.VMEM((NBUF2, CH2, HID), _f32),
            pltpu.VMEM((NBUF2, CH2, 16), _f32),
            pltpu.VMEM((NBUF2, CH2, 16), _f32),
            pltpu.VMEM_SHARED((NPAD, HID), _f32),
            pltpu.SemaphoreType.DMA,
            pltpu.SemaphoreType.DMA,
            pltpu.SemaphoreType.DMA,
            pltpu.SemaphoreType.DMA,
            pltpu.SemaphoreType.DMA,
            pltpu.SemaphoreType.DMA,
            pltpu.SemaphoreType.DMA,
            pltpu.SemaphoreType.DMA,
        ],
        compiler_params=pltpu.CompilerParams(use_tc_tiling_on_sc=False),
    )
    return k(ei_flat, ex, hh_flat, dA)


def _layer_norm(m, s, b):
    mu = jnp.mean(m, axis=-1, keepdims=True)
    var = jnp.mean((m - mu) ** 2, axis=-1, keepdims=True)
    return (m - mu) * lax.rsqrt(var + 1e-5) * s + b


def _elu(y):
    return jnp.where(y > 0, y, jnp.exp(y) - 1.0)


def _tcc_body(msg_ref, h0_ref, pe_ref, ln1_s_ref, ln1_b_ref,
              wqT_ref, bq_ref, wkT_ref, bk_ref, wvT_ref, bv_ref,
              eh_ref, eexp_ref, woutT_ref, bout_ref,
              ln2_s_ref, ln2_b_ref, wcT_ref, bc_ref, out_ref):
    seqs = []
    for t in range(T):
        m = msg_ref[0, t] + msg_ref[1, t] + h0_ref[t]
        y = _layer_norm(m, ln1_s_ref[0], ln1_b_ref[0])
        seqs.append(_elu(y) + pe_ref[t])
    x3 = seqs[T - 1]
    q3 = jnp.dot(x3, wqT_ref[...], preferred_element_type=_f32) + bq_ref[0]
    aw = []
    vs = []
    for t in range(T):
        kt = jnp.dot(seqs[t], wkT_ref[...],
                     preferred_element_type=_f32) + bk_ref[0]
        vs.append(jnp.dot(seqs[t], wvT_ref[...],
                          preferred_element_type=_f32) + bv_ref[0])
        aw.append(jnp.dot(q3 * kt, eh_ref[...],
                          preferred_element_type=_f32))
    mx = jnp.maximum(jnp.maximum(aw[0], aw[1]), jnp.maximum(aw[2], aw[3]))
    es = [jnp.exp(a - mx) for a in aw]
    den = es[0] + es[1] + es[2] + es[3]
    ao = None
    for t in range(T):
        w = es[t] / den
        wex = jnp.dot(w, eexp_ref[...], preferred_element_type=_f32)
        ao = wex * vs[t] if ao is None else ao + wex * vs[t]
    out = jnp.dot(ao, woutT_ref[...], preferred_element_type=_f32) + bout_ref[0]
    y2 = _layer_norm(x3 + out, ln2_s_ref[0], ln2_b_ref[0])
    z = _elu(y2)
    out_ref[...] = jnp.dot(z, wcT_ref[...],
                           preferred_element_type=_f32) + bc_ref[0]


def _run_tcc(msg, h0, pe, ln1_s2, ln1_b2, wqT, bq2, wkT, bk2, wvT, bv2,
             eh, eexp, woutT, bout2, ln2_s2, ln2_b2, wcT_pad, bc2_pad):
    full = lambda s: pl.BlockSpec(s, lambda nb: tuple(0 for _ in s))
    return pl.pallas_call(
        _tcc_body,
        grid=(NB,),
        in_specs=[
            pl.BlockSpec((2, T, BLK, HID), lambda nb: (0, 0, nb, 0)),
            pl.BlockSpec((T, BLK, HID), lambda nb: (0, nb, 0)),
            full((T, HID)),
            full((1, HID)), full((1, HID)),
            full((HID, HID)), full((1, HID)),
            full((HID, HID)), full((1, HID)),
            full((HID, HID)), full((1, HID)),
            full((HID, 16)), full((16, HID)),
            full((HID, HID)), full((1, HID)),
            full((1, HID)), full((1, HID)),
            full((HID, HID)), full((1, HID)),
        ],
        out_specs=pl.BlockSpec((BLK, HID), lambda nb: (nb, 0)),
        out_shape=jax.ShapeDtypeStruct((N, HID), _f32),
    )(msg, h0, pe, ln1_s2, ln1_b2, wqT, bq2, wkT, bk2, wvT, bv2,
      eh, eexp, woutT, bout2, ln2_s2, ln2_b2, wcT_pad, bc2_pad)


def _pos_enc():
    pos = jnp.arange(T, dtype=_f32)[:, None]
    div = jnp.exp(jnp.arange(0, HID, 2, dtype=_f32)
                  * (-math.log(10000.0) / HID))
    pe = jnp.zeros((T, HID), dtype=_f32)
    pe = pe.at[:, 0::2].set(jnp.sin(pos * div))
    pe = pe.at[:, 1::2].set(jnp.cos(pos * div))
    return pe


def kernel(x, edge_index, W_in, b_in, W_gat, a_gat, ln1_s, ln1_b, Wqkv, bqkv,
           Wout, bout, ln2_s, ln2_b, Wc, bc):
    winT = W_in.T
    wgatT = W_gat.T
    a1 = a_gat[0, :DH]
    a2 = a_gat[0, DH:]
    sel1 = jnp.kron(jnp.eye(H, dtype=_f32), a1[:, None])
    sel2 = jnp.kron(jnp.eye(H, dtype=_f32), a2[:, None])
    wsrc = wgatT @ jnp.concatenate([sel1, sel1], axis=1)
    wdst = wgatT @ jnp.concatenate([sel2, sel2], axis=1)

    ei = edge_index.astype(_i32)
    ei = jnp.pad(ei, ((0, 0), (0, 0), (0, EPAD - E)))
    ei_flat = ei.reshape(T * 2 * EPAD)

    h0, hh, ss, sd = _run_tca(x, winT, b_in[None, :], wgatT, wsrc, wdst)
    ssrc16 = ss.reshape(T * N, 16)
    sdst16 = sd.reshape(T * N, 16)
    hh_flat = hh.reshape(T * N, HID)

    cm = jnp.max(ss, axis=1) + jnp.max(sd, axis=1)
    cmax = jnp.where(cm > 0, cm, 0.2 * cm)

    den, ex = _run_sc1(ei_flat, ssrc16, sdst16, cmax)
    dsum = _run_densum(den.reshape(2, TNPAD, 16))

    msg = _run_sc2(ei_flat, ex, hh_flat, dsum)
    msg = msg.reshape(2, T, NPAD, HID)

    qs, ks_, vs_ = [], [], []
    for h in range(H):
        qs.append(Wqkv[h * 3 * DH: h * 3 * DH + DH])
        ks_.append(Wqkv[h * 3 * DH + DH: h * 3 * DH + 2 * DH])
        vs_.append(Wqkv[h * 3 * DH + 2 * DH: h * 3 * DH + 3 * DH])
    wq = jnp.concatenate(qs, axis=0)
    wk = jnp.concatenate(ks_, axis=0)
    wv = jnp.concatenate(vs_, axis=0)
    bqkv3 = bqkv.reshape(H, 3 * DH)
    bq = bqkv3[:, :DH].reshape(HID)
    bk = bqkv3[:, DH:2 * DH].reshape(HID)
    bv = bqkv3[:, 2 * DH:].reshape(HID)

    eh = jnp.kron(jnp.eye(H, dtype=_f32), jnp.ones((DH, 1), _f32))
    eh16 = jnp.concatenate([eh, eh], axis=1) / math.sqrt(DH)
    eexp = jnp.concatenate([eh, eh], axis=1).T * 0.5

    wcT_pad = jnp.zeros((HID, HID), _f32).at[:, :NCLS].set(Wc.T)
    bc_pad = jnp.zeros((HID,), _f32).at[:NCLS].set(bc)

    logits = _run_tcc(msg, h0, _pos_enc(), ln1_s[None, :], ln1_b[None, :],
                      wq.T, bq[None, :], wk.T, bk[None, :], wv.T, bv[None, :],
                      eh16, eexp, Wout.T, bout[None, :],
                      ln2_s[None, :], ln2_b[None, :], wcT_pad, bc_pad[None, :])
    return logits[:, :NCLS]
